# Initial kernel scaffold; baseline (speedup 1.0000x reference)
import jax
import jax.numpy as jnp
from jax.experimental import pallas as pl


def kernel(x, edge_index, W_aff, b_aff, W0a, b0a, W0b, b0b, bn0_g, bn0_b, W1a, b1a, W1b, b1b, bn1_g, bn1_b, W_in, b_in, Wc0a, bc0a, Wc0b, bc0b, bnc0_g, bnc0_b, Wc1a, bc1a, Wc1b, bc1b, bnc1_g, bnc1_b):
    raise NotImplementedError("write your pallas kernel here")


# speedup vs baseline: 1.0056x; 1.0056x over previous
import jax
import jax.numpy as jnp
from jax.experimental import pallas as pl
from jax.experimental.pallas import tpu as pltpu

_NC = 4


def _bn_relu_kernel(h_ref, g_ref, b_ref, o_ref):
    g = g_ref[...] * (1.0 / jnp.sqrt(1.0 + 1e-5))
    o_ref[...] = jnp.maximum(h_ref[...] * g + b_ref[...], 0.0)


def _bn_relu(h, g, b):
    return pl.pallas_call(
        _bn_relu_kernel,
        out_shape=jax.ShapeDtypeStruct(h.shape, h.dtype),
    )(h, g.reshape(1, -1), b.reshape(1, -1))


def kernel(x, edge_index, W_aff, b_aff, W0a, b0a, W0b, b0b, bn0_g, bn0_b,
           W1a, b1a, W1b, b1b, bn1_g, bn1_b, W_in, b_in,
           Wc0a, bc0a, Wc0b, bc0b, bnc0_g, bnc0_b,
           Wc1a, bc1a, Wc1b, bc1b, bnc1_g, bnc1_b):
    src, dst = edge_index[0], edge_index[1]

    def gin(h, ew, Wa, ba, Wb, bb):
        msg = h[src] * ew[:, None]
        agg = jnp.zeros_like(h).at[dst].add(msg)
        u = jax.nn.relu(jnp.dot(h + agg, Wa) + ba)
        return jnp.dot(u, Wb) + bb

    z = jax.nn.relu(jnp.dot(x, W_aff) + b_aff)
    zc = jnp.split(z, _NC, axis=-1)
    logits = jnp.stack([jnp.sum(zk[src] * zk[dst], axis=1) for zk in zc])
    ew_all = jax.nn.softmax(logits, axis=0)
    h1_list, h2_list = [], []
    for k in range(_NC):
        ew = ew_all[k]
        h1 = _bn_relu(gin(x, ew, W0a, b0a, W0b, b0b), bn0_g, bn0_b)
        h2 = _bn_relu(gin(h1, ew, W1a, b1a, W1b, b1b), bn1_g, bn1_b)
        h1_list.append(h1)
        h2_list.append(h2)
    h = (jnp.dot(x, W_in) + b_in) + jnp.concatenate(h1_list, axis=-1) + jnp.concatenate(h2_list, axis=-1)
    ones = jnp.ones((src.shape[0],), x.dtype)
    g1 = _bn_relu(gin(h, ones, Wc0a, bc0a, Wc0b, bc0b), bnc0_g, bnc0_b)
    g2 = _bn_relu(gin(g1, ones, Wc1a, bc1a, Wc1b, bc1b), bnc1_g, bnc1_b)
    return g2


# speedup vs baseline: 1.1855x; 1.1790x over previous
import functools

import jax
import jax.numpy as jnp
from jax import lax
from jax.experimental import pallas as pl
from jax.experimental.pallas import tpu as pltpu
from jax.experimental.pallas import tpu_sc as plsc

N = 10000
E = 320000
D = 128
NCOM = 4
CD = 32

NCOR = 2
NSUB = 16
LANE = 16

KW = 2
WINE = KW * 128
EP = 327680
NP = N + 112
RPS = NP // NSUB


def _sc_mesh():
    return plsc.VectorSubcoreMesh(core_axis_name="c", subcore_axis_name="s")


def _unit_agg_body(tab, ei_r, zeros, out, src_i, dst_i, rows_v, acc_sh, sem):
    c = lax.axis_index("c")
    s = lax.axis_index("s")
    pltpu.sync_copy(zeros.at[pl.ds(s * RPS, RPS)],
                    acc_sh.at[pl.ds(s * RPS, RPS)])
    plsc.subcore_barrier()

    wid = s * NCOR + c
    nw = EP // 32 // WINE

    def window(w, carry):
        row0 = wid * (nw * KW) + w * KW
        pltpu.sync_copy(ei_r.at[0, pl.ds(row0, KW)], src_i)
        pltpu.sync_copy(ei_r.at[1, pl.ds(row0, KW)], dst_i)
        for j in range(KW):
            pltpu.async_copy(tab.at[src_i.at[j]],
                             rows_v.at[pl.ds(j * 128, 128)], sem).wait()
        for j in range(KW):
            pltpu.sync_copy(rows_v.at[pl.ds(j * 128, 128)],
                            acc_sh.at[dst_i.at[j]], add=True)
        return carry

    lax.fori_loop(0, nw, window, 0, unroll=False)
    plsc.subcore_barrier()
    pltpu.sync_copy(acc_sh.at[pl.ds(s * RPS, RPS)],
                    out.at[c, pl.ds(s * RPS, RPS)])


@jax.jit
def _sc_unit_agg(tab, ei_r, zeros):
    return pl.kernel(
        _unit_agg_body,
        out_type=jax.ShapeDtypeStruct((2, NP, 128), jnp.float32),
        mesh=_sc_mesh(),
        scratch_types=[
            pltpu.VMEM((KW, 128), jnp.int32),
            pltpu.VMEM((KW, 128), jnp.int32),
            pltpu.VMEM((WINE, 128), jnp.float32),
            pltpu.VMEM_SHARED((NP, 128), jnp.float32),
            pltpu.SemaphoreType.DMA,
        ],
        compiler_params=pltpu.CompilerParams(use_tc_tiling_on_sc=False),
        name="sc_unit_agg",
    )(tab, ei_r, zeros)


def _bn(h, g, b):
    return h * (g / jnp.sqrt(1.0 + 1e-5)) + b


def kernel(x, edge_index, W_aff, b_aff, W0a, b0a, W0b, b0b, bn0_g, bn0_b,
           W1a, b1a, W1b, b1b, bn1_g, bn1_b, W_in, b_in,
           Wc0a, bc0a, Wc0b, bc0b, bnc0_g, bnc0_b,
           Wc1a, bc1a, Wc1b, bc1b, bnc1_g, bnc1_b):
    src, dst = edge_index[0], edge_index[1]

    npad = EP - E
    pad_lane = (jnp.arange(npad, dtype=jnp.int32) % 112)
    ei_r = jnp.concatenate([
        jnp.stack([src, dst]),
        jnp.stack([pad_lane, N + pad_lane]),
    ], axis=1).reshape(2, EP // 128, 128)
    zerosNP = jnp.zeros((NP, 128), jnp.float32)

    def gin(h, ew, Wa, ba, Wb, bb):
        msg = h[src] * ew[:, None]
        agg = jnp.zeros_like(h).at[dst].add(msg)
        u = jax.nn.relu(jnp.dot(h + agg, Wa) + ba)
        return jnp.dot(u, Wb) + bb

    z = jax.nn.relu(jnp.dot(x, W_aff) + b_aff)
    zc = jnp.split(z, NCOM, axis=-1)
    logits = jnp.stack([jnp.sum(zk[src] * zk[dst], axis=1) for zk in zc])
    ew_all = jax.nn.softmax(logits, axis=0)
    h1_list, h2_list = [], []
    for k in range(NCOM):
        ew = ew_all[k]
        h1 = jax.nn.relu(_bn(gin(x, ew, W0a, b0a, W0b, b0b), bn0_g, bn0_b))
        h2 = jax.nn.relu(_bn(gin(h1, ew, W1a, b1a, W1b, b1b), bn1_g, bn1_b))
        h1_list.append(h1)
        h2_list.append(h2)
    h = (jnp.dot(x, W_in) + b_in) + jnp.concatenate(h1_list, axis=-1) \
        + jnp.concatenate(h2_list, axis=-1)

    hw = jnp.dot(h, Wc0a)
    p1 = _sc_unit_agg(hw, ei_r, zerosNP)
    agg1 = (p1[0] + p1[1])[:N]
    u = jax.nn.relu(hw + agg1 + bc0a)
    g1 = jax.nn.relu(_bn(jnp.dot(u, Wc0b) + bc0b, bnc0_g, bnc0_b))

    gw = jnp.dot(g1, Wc1a)
    p2 = _sc_unit_agg(gw, ei_r, zerosNP)
    agg2 = (p2[0] + p2[1])[:N]
    u2 = jax.nn.relu(gw + agg2 + bc1a)
    g2 = jax.nn.relu(_bn(jnp.dot(u2, Wc1b) + bc1b, bnc1_g, bnc1_b))
    return g2


# speedup vs baseline: 1.9704x; 1.6620x over previous
import functools

import jax
import jax.numpy as jnp
from jax import lax
from jax.experimental import pallas as pl
from jax.experimental.pallas import tpu as pltpu
from jax.experimental.pallas import tpu_sc as plsc

N = 10000
E = 320000
D = 128
NCOM = 4
CD = 32

NCOR = 2
NSUB = 16
LANE = 16

KW = 2
WINE = KW * 128
EP = 327680
NP = N + 112
RPS = NP // NSUB


def _sc_mesh():
    return plsc.VectorSubcoreMesh(core_axis_name="c", subcore_axis_name="s")


def _unit_agg_body(tab, ei_r, zeros, out, src_i, dst_i, rows_v, acc_sh, sem):
    c = lax.axis_index("c")
    s = lax.axis_index("s")
    pltpu.sync_copy(zeros.at[pl.ds(s * RPS, RPS)],
                    acc_sh.at[pl.ds(s * RPS, RPS)])
    plsc.subcore_barrier()

    wid = s * NCOR + c
    nw = EP // 32 // WINE

    def window(w, carry):
        row0 = wid * (nw * KW) + w * KW
        pltpu.sync_copy(ei_r.at[0, pl.ds(row0, KW)], src_i)
        pltpu.sync_copy(ei_r.at[1, pl.ds(row0, KW)], dst_i)
        for j in range(KW):
            pltpu.async_copy(tab.at[src_i.at[j]],
                             rows_v.at[pl.ds(j * 128, 128)], sem).wait()
        for j in range(KW):
            pltpu.sync_copy(rows_v.at[pl.ds(j * 128, 128)],
                            acc_sh.at[dst_i.at[j]], add=True)
        return carry

    lax.fori_loop(0, nw, window, 0, unroll=False)
    plsc.subcore_barrier()
    pltpu.sync_copy(acc_sh.at[pl.ds(s * RPS, RPS)],
                    out.at[c, pl.ds(s * RPS, RPS)])


@jax.jit
def _sc_unit_agg(tab, ei_r, zeros):
    return pl.kernel(
        _unit_agg_body,
        out_type=jax.ShapeDtypeStruct((2, NP, 128), jnp.float32),
        mesh=_sc_mesh(),
        scratch_types=[
            pltpu.VMEM((KW, 128), jnp.int32),
            pltpu.VMEM((KW, 128), jnp.int32),
            pltpu.VMEM((WINE, 128), jnp.float32),
            pltpu.VMEM_SHARED((NP, 128), jnp.float32),
            pltpu.SemaphoreType.DMA,
        ],
        compiler_params=pltpu.CompilerParams(use_tc_tiling_on_sc=False),
        name="sc_unit_agg",
    )(tab, ei_r, zeros)


def _make_wagg_body(tw, wine):
    kw = wine // 128

    def body(tab, ei_r, lg, zeros, out, src_i, dst_i, rows_v, msg_v, ew_v,
             acc_sh, sem):
        c = lax.axis_index("c")
        s = lax.axis_index("s")
        pltpu.sync_copy(zeros.at[pl.ds(s * RPS, RPS)],
                        acc_sh.at[pl.ds(s * RPS, RPS)])
        plsc.subcore_barrier()

        wid = s * NCOR + c
        nw = EP // 32 // wine

        def window(w, carry):
            e0 = wid * (nw * wine) + w * wine
            row0 = e0 // 128
            pltpu.sync_copy(ei_r.at[0, pl.ds(row0, kw)], src_i)
            pltpu.sync_copy(ei_r.at[1, pl.ds(row0, kw)], dst_i)
            for j in range(kw):
                pltpu.async_copy(tab.at[src_i.at[j]],
                                 rows_v.at[pl.ds(j * 128, 128)], sem).wait()
            pltpu.sync_copy(lg.at[:, pl.ds(e0, wine)],
                            ew_v.at[:, pl.ds(0, wine)])
            for g in range(wine // 16):
                sl = pl.ds(g * 16, 16)
                l0, l1, l2, l3 = (ew_v[0, sl], ew_v[1, sl], ew_v[2, sl],
                                  ew_v[3, sl])
                m = jnp.maximum(jnp.maximum(l0, l1), jnp.maximum(l2, l3))
                x0, x1 = jnp.exp(l0 - m), jnp.exp(l1 - m)
                x2, x3 = jnp.exp(l2 - m), jnp.exp(l3 - m)
                inv = 1.0 / (x0 + x1 + x2 + x3)
                ew_v[0, sl] = x0 * inv
                ew_v[1, sl] = x1 * inv
                ew_v[2, sl] = x2 * inv
                ew_v[3, sl] = x3 * inv

            def edge(e, carry2):
                if tw == 32:
                    ya = rows_v[e, pl.ds(0, 16)]
                    yb = rows_v[e, pl.ds(16, 16)]
                    for k in range(NCOM):
                        wk = ew_v[k, pl.ds(e, 16)][0]
                        msg_v[e, pl.ds(k * 32, 16)] = ya * wk
                        msg_v[e, pl.ds(k * 32 + 16, 16)] = yb * wk
                else:
                    for k in range(NCOM):
                        wk = ew_v[k, pl.ds(e, 16)][0]
                        a = pl.ds(k * 32, 16)
                        b = pl.ds(k * 32 + 16, 16)
                        msg_v[e, a] = rows_v[e, a] * wk
                        msg_v[e, b] = rows_v[e, b] * wk
                return carry2

            lax.fori_loop(0, wine, edge, 0, unroll=False)
            for j in range(kw):
                pltpu.sync_copy(msg_v.at[pl.ds(j * 128, 128)],
                                acc_sh.at[dst_i.at[j]], add=True)
            return carry

        lax.fori_loop(0, nw, window, 0, unroll=False)
        plsc.subcore_barrier()
        pltpu.sync_copy(acc_sh.at[pl.ds(s * RPS, RPS)],
                        out.at[c, pl.ds(s * RPS, RPS)])

    return body


def _make_wagg(tw, wine):
    kw = wine // 128
    rows_shape = (wine, tw) if tw == 32 else (wine, 128)

    @jax.jit
    def run(tab, ei_r, lg, zeros):
        return pl.kernel(
            _make_wagg_body(tw, wine),
            out_type=jax.ShapeDtypeStruct((2, NP, 128), jnp.float32),
            mesh=_sc_mesh(),
            scratch_types=[
                pltpu.VMEM((kw, 128), jnp.int32),
                pltpu.VMEM((kw, 128), jnp.int32),
                pltpu.VMEM(rows_shape, jnp.float32),
                pltpu.VMEM((wine, 128), jnp.float32),
                pltpu.VMEM((4, wine + 16), jnp.float32),
                pltpu.VMEM_SHARED((NP, 128), jnp.float32),
                pltpu.SemaphoreType.DMA,
            ],
            compiler_params=pltpu.CompilerParams(use_tc_tiling_on_sc=False),
            name=f"sc_wagg{tw}",
        )(tab, ei_r, lg, zeros)

    return run


_sc_wagg32 = _make_wagg(32, 128)
_sc_wagg128 = _make_wagg(128, 128)


def _bn(h, g, b):
    return h * (g / jnp.sqrt(1.0 + 1e-5)) + b


def kernel(x, edge_index, W_aff, b_aff, W0a, b0a, W0b, b0b, bn0_g, bn0_b,
           W1a, b1a, W1b, b1b, bn1_g, bn1_b, W_in, b_in,
           Wc0a, bc0a, Wc0b, bc0b, bnc0_g, bnc0_b,
           Wc1a, bc1a, Wc1b, bc1b, bnc1_g, bnc1_b):
    src, dst = edge_index[0], edge_index[1]

    npad = EP - E
    pad_lane = (jnp.arange(npad, dtype=jnp.int32) % 112)
    ei_r = jnp.concatenate([
        jnp.stack([src, dst]),
        jnp.stack([pad_lane, N + pad_lane]),
    ], axis=1).reshape(2, EP // 128, 128)
    zerosNP = jnp.zeros((NP, 128), jnp.float32)

    z = jax.nn.relu(jnp.dot(x, W_aff) + b_aff)
    zc = jnp.split(z, NCOM, axis=-1)
    logits = jnp.stack([jnp.sum(zk[src] * zk[dst], axis=1) for zk in zc])
    lg = jnp.pad(logits, ((0, 0), (0, npad)))

    y = jnp.dot(x, W0a)
    pB = _sc_wagg32(y, ei_r, lg, zerosNP)
    S = (pB[0] + pB[1])[:N].reshape(N, NCOM, CD)
    u0 = jax.nn.relu(y[:, None, :] + S + b0a)
    h1 = jax.nn.relu(_bn(jnp.einsum("nkf,fg->nkg", u0, W0b) + b0b,
                         bn0_g, bn0_b))

    t = jnp.einsum("nkf,fg->nkg", h1, W1a)
    pC = _sc_wagg128(t.reshape(N, D), ei_r, lg, zerosNP)
    A1 = (pC[0] + pC[1])[:N].reshape(N, NCOM, CD)
    u1 = jax.nn.relu(t + A1 + b1a)
    h2 = jax.nn.relu(_bn(jnp.einsum("nkf,fg->nkg", u1, W1b) + b1b,
                         bn1_g, bn1_b))

    h = (jnp.dot(x, W_in) + b_in) + h1.reshape(N, D) + h2.reshape(N, D)

    hw = jnp.dot(h, Wc0a)
    p1 = _sc_unit_agg(hw, ei_r, zerosNP)
    agg1 = (p1[0] + p1[1])[:N]
    u = jax.nn.relu(hw + agg1 + bc0a)
    g1 = jax.nn.relu(_bn(jnp.dot(u, Wc0b) + bc0b, bnc0_g, bnc0_b))

    gw = jnp.dot(g1, Wc1a)
    p2 = _sc_unit_agg(gw, ei_r, zerosNP)
    agg2 = (p2[0] + p2[1])[:N]
    u2 = jax.nn.relu(gw + agg2 + bc1a)
    g2 = jax.nn.relu(_bn(jnp.dot(u2, Wc1b) + bc1b, bnc1_g, bnc1_b))
    return g2


# speedup vs baseline: 4.5513x; 2.3099x over previous
import functools

import jax
import jax.numpy as jnp
from jax import lax
from jax.experimental import pallas as pl
from jax.experimental.pallas import tpu as pltpu
from jax.experimental.pallas import tpu_sc as plsc

N = 10000
E = 320000
D = 128
NCOM = 4
CD = 32

NCOR = 2
NSUB = 16
LANE = 16

KW = 2
WINE = KW * 128
EP = 327680
NP = N + 112
RPS = NP // NSUB


def _sc_mesh():
    return plsc.VectorSubcoreMesh(core_axis_name="c", subcore_axis_name="s")


def _unit_agg_body(tab, ei_r, zeros, out, src_i, dst_i, rows_v, acc_sh, sem):
    c = lax.axis_index("c")
    s = lax.axis_index("s")
    pltpu.sync_copy(zeros.at[pl.ds(s * RPS, RPS)],
                    acc_sh.at[pl.ds(s * RPS, RPS)])
    plsc.subcore_barrier()

    wid = s * NCOR + c
    nw = EP // 32 // WINE

    def window(w, carry):
        row0 = wid * (nw * KW) + w * KW
        pltpu.sync_copy(ei_r.at[0, pl.ds(row0, KW)], src_i)
        pltpu.sync_copy(ei_r.at[1, pl.ds(row0, KW)], dst_i)
        for j in range(KW):
            pltpu.async_copy(tab.at[src_i.at[j]],
                             rows_v.at[pl.ds(j * 128, 128)], sem).wait()
        for j in range(KW):
            pltpu.sync_copy(rows_v.at[pl.ds(j * 128, 128)],
                            acc_sh.at[dst_i.at[j]], add=True)
        return carry

    lax.fori_loop(0, nw, window, 0, unroll=False)
    plsc.subcore_barrier()
    pltpu.sync_copy(acc_sh.at[pl.ds(s * RPS, RPS)],
                    out.at[c, pl.ds(s * RPS, RPS)])


@jax.jit
def _sc_unit_agg(tab, ei_r, zeros):
    return pl.kernel(
        _unit_agg_body,
        out_type=jax.ShapeDtypeStruct((2, NP, 128), jnp.float32),
        mesh=_sc_mesh(),
        scratch_types=[
            pltpu.VMEM((KW, 128), jnp.int32),
            pltpu.VMEM((KW, 128), jnp.int32),
            pltpu.VMEM((WINE, 128), jnp.float32),
            pltpu.VMEM_SHARED((NP, 128), jnp.float32),
            pltpu.SemaphoreType.DMA,
        ],
        compiler_params=pltpu.CompilerParams(use_tc_tiling_on_sc=False),
        name="sc_unit_agg",
    )(tab, ei_r, zeros)


def _make_wagg_body(tw, wine):
    kw = wine // 128

    def body(tab, ei_r, lg, zeros, out, src_i, dst_i, rows_v, msg_v, ew_v,
             acc_sh, sem):
        c = lax.axis_index("c")
        s = lax.axis_index("s")
        pltpu.sync_copy(zeros.at[pl.ds(s * RPS, RPS)],
                        acc_sh.at[pl.ds(s * RPS, RPS)])
        plsc.subcore_barrier()

        wid = s * NCOR + c
        nw = EP // 32 // wine

        def window(w, carry):
            e0 = wid * (nw * wine) + w * wine
            row0 = e0 // 128
            pltpu.sync_copy(ei_r.at[0, pl.ds(row0, kw)], src_i)
            pltpu.sync_copy(ei_r.at[1, pl.ds(row0, kw)], dst_i)
            for j in range(kw):
                pltpu.async_copy(tab.at[src_i.at[j]],
                                 rows_v.at[pl.ds(j * 128, 128)], sem).wait()
            pltpu.sync_copy(lg.at[:, pl.ds(e0, wine)],
                            ew_v.at[:, pl.ds(0, wine)])
            for g in range(wine // 16):
                sl = pl.ds(g * 16, 16)
                l0, l1, l2, l3 = (ew_v[0, sl], ew_v[1, sl], ew_v[2, sl],
                                  ew_v[3, sl])
                m = jnp.maximum(jnp.maximum(l0, l1), jnp.maximum(l2, l3))
                x0, x1 = jnp.exp(l0 - m), jnp.exp(l1 - m)
                x2, x3 = jnp.exp(l2 - m), jnp.exp(l3 - m)
                inv = 1.0 / (x0 + x1 + x2 + x3)
                ew_v[0, sl] = x0 * inv
                ew_v[1, sl] = x1 * inv
                ew_v[2, sl] = x2 * inv
                ew_v[3, sl] = x3 * inv

            def edge(e, carry2):
                if tw == 32:
                    ya = rows_v[e, pl.ds(0, 16)]
                    yb = rows_v[e, pl.ds(16, 16)]
                    for k in range(NCOM):
                        wk = ew_v[k, pl.ds(e, 16)][0]
                        msg_v[e, pl.ds(k * 32, 16)] = ya * wk
                        msg_v[e, pl.ds(k * 32 + 16, 16)] = yb * wk
                else:
                    for k in range(NCOM):
                        wk = ew_v[k, pl.ds(e, 16)][0]
                        a = pl.ds(k * 32, 16)
                        b = pl.ds(k * 32 + 16, 16)
                        msg_v[e, a] = rows_v[e, a] * wk
                        msg_v[e, b] = rows_v[e, b] * wk
                return carry2

            lax.fori_loop(0, wine, edge, 0, unroll=False)
            for j in range(kw):
                pltpu.sync_copy(msg_v.at[pl.ds(j * 128, 128)],
                                acc_sh.at[dst_i.at[j]], add=True)
            return carry

        lax.fori_loop(0, nw, window, 0, unroll=False)
        plsc.subcore_barrier()
        pltpu.sync_copy(acc_sh.at[pl.ds(s * RPS, RPS)],
                        out.at[c, pl.ds(s * RPS, RPS)])

    return body


def _make_wagg(tw, wine):
    kw = wine // 128
    rows_shape = (wine, tw) if tw == 32 else (wine, 128)

    @jax.jit
    def run(tab, ei_r, lg, zeros):
        return pl.kernel(
            _make_wagg_body(tw, wine),
            out_type=jax.ShapeDtypeStruct((2, NP, 128), jnp.float32),
            mesh=_sc_mesh(),
            scratch_types=[
                pltpu.VMEM((kw, 128), jnp.int32),
                pltpu.VMEM((kw, 128), jnp.int32),
                pltpu.VMEM(rows_shape, jnp.float32),
                pltpu.VMEM((wine, 128), jnp.float32),
                pltpu.VMEM((4, wine + 16), jnp.float32),
                pltpu.VMEM_SHARED((NP, 128), jnp.float32),
                pltpu.SemaphoreType.DMA,
            ],
            compiler_params=pltpu.CompilerParams(use_tc_tiling_on_sc=False),
            name=f"sc_wagg{tw}",
        )(tab, ei_r, lg, zeros)

    return run


_sc_wagg32 = _make_wagg(32, 128)
_sc_wagg128 = _make_wagg(128, 128)


WINL = 256
KL = WINL // 128


def _logits_body(z, ei_r, out, src_i, dst_i, zs_v, zd_v, lgo_v, sem):
    c = lax.axis_index("c")
    s = lax.axis_index("s")
    wid = s * NCOR + c
    nw = EP // 32 // WINL
    lanes = jnp.arange(16, dtype=jnp.int32)

    def window(w, carry):
        e0 = wid * (nw * WINL) + w * WINL
        row0 = e0 // 128
        pltpu.sync_copy(ei_r.at[0, pl.ds(row0, KL)], src_i)
        pltpu.sync_copy(ei_r.at[1, pl.ds(row0, KL)], dst_i)
        for j in range(KL):
            pltpu.async_copy(z.at[src_i.at[j]],
                             zs_v.at[pl.ds(j * 128, 128)], sem).wait()
            pltpu.async_copy(z.at[dst_i.at[j]],
                             zd_v.at[pl.ds(j * 128, 128)], sem).wait()

        def gblock(g, carry2):
            rows = g * 16 + lanes
            for k in range(NCOM):
                def dstep(d, acc):
                    cols = jnp.full((16,), d, jnp.int32)
                    vs = plsc.load_gather(zs_v, [rows, cols])
                    vd = plsc.load_gather(zd_v, [rows, cols])
                    return acc + vs * vd

                acc = lax.fori_loop(k * 32, k * 32 + 32, dstep,
                                    jnp.zeros((16,), jnp.float32), unroll=4)
                lgo_v[k, pl.ds(g * 16, 16)] = acc
            return carry2

        lax.fori_loop(0, WINL // 16, gblock, 0, unroll=False)
        pltpu.sync_copy(lgo_v, out.at[:, pl.ds(e0, WINL)])
        return carry

    lax.fori_loop(0, nw, window, 0, unroll=False)


@jax.jit
def _sc_logits(z, ei_r):
    return pl.kernel(
        _logits_body,
        out_type=jax.ShapeDtypeStruct((4, EP), jnp.float32),
        mesh=_sc_mesh(),
        scratch_types=[
            pltpu.VMEM((KL, 128), jnp.int32),
            pltpu.VMEM((KL, 128), jnp.int32),
            pltpu.VMEM((WINL, 128), jnp.float32),
            pltpu.VMEM((WINL, 128), jnp.float32),
            pltpu.VMEM((4, WINL), jnp.float32),
            pltpu.SemaphoreType.DMA,
        ],
        compiler_params=pltpu.CompilerParams(use_tc_tiling_on_sc=False,
                                             needs_layout_passes=False),
        name="sc_logits",
    )(z, ei_r)


def _bn(h, g, b):
    return h * (g / jnp.sqrt(1.0 + 1e-5)) + b


def kernel(x, edge_index, W_aff, b_aff, W0a, b0a, W0b, b0b, bn0_g, bn0_b,
           W1a, b1a, W1b, b1b, bn1_g, bn1_b, W_in, b_in,
           Wc0a, bc0a, Wc0b, bc0b, bnc0_g, bnc0_b,
           Wc1a, bc1a, Wc1b, bc1b, bnc1_g, bnc1_b):
    src, dst = edge_index[0], edge_index[1]

    npad = EP - E
    pad_lane = (jnp.arange(npad, dtype=jnp.int32) % 112)
    ei_r = jnp.concatenate([
        jnp.stack([src, dst]),
        jnp.stack([pad_lane, N + pad_lane]),
    ], axis=1).reshape(2, EP // 128, 128)
    zerosNP = jnp.zeros((NP, 128), jnp.float32)

    z = jax.nn.relu(jnp.dot(x, W_aff) + b_aff)
    lg = _sc_logits(z, ei_r)

    y = jnp.dot(x, W0a)
    pB = _sc_wagg32(y, ei_r, lg, zerosNP)
    S = (pB[0] + pB[1])[:N].reshape(N, NCOM, CD)
    u0 = jax.nn.relu(y[:, None, :] + S + b0a)
    h1 = jax.nn.relu(_bn(jnp.einsum("nkf,fg->nkg", u0, W0b) + b0b,
                         bn0_g, bn0_b))

    t = jnp.einsum("nkf,fg->nkg", h1, W1a)
    pC = _sc_wagg128(t.reshape(N, D), ei_r, lg, zerosNP)
    A1 = (pC[0] + pC[1])[:N].reshape(N, NCOM, CD)
    u1 = jax.nn.relu(t + A1 + b1a)
    h2 = jax.nn.relu(_bn(jnp.einsum("nkf,fg->nkg", u1, W1b) + b1b,
                         bn1_g, bn1_b))

    h = (jnp.dot(x, W_in) + b_in) + h1.reshape(N, D) + h2.reshape(N, D)

    hw = jnp.dot(h, Wc0a)
    p1 = _sc_unit_agg(hw, ei_r, zerosNP)
    agg1 = (p1[0] + p1[1])[:N]
    u = jax.nn.relu(hw + agg1 + bc0a)
    g1 = jax.nn.relu(_bn(jnp.dot(u, Wc0b) + bc0b, bnc0_g, bnc0_b))

    gw = jnp.dot(g1, Wc1a)
    p2 = _sc_unit_agg(gw, ei_r, zerosNP)
    agg2 = (p2[0] + p2[1])[:N]
    u2 = jax.nn.relu(gw + agg2 + bc1a)
    g2 = jax.nn.relu(_bn(jnp.dot(u2, Wc1b) + bc1b, bnc1_g, bnc1_b))
    return g2


# speedup vs baseline: 4.5951x; 1.0096x over previous
import functools

import jax
import jax.numpy as jnp
from jax import lax
from jax.experimental import pallas as pl
from jax.experimental.pallas import tpu as pltpu
from jax.experimental.pallas import tpu_sc as plsc

N = 10000
E = 320000
D = 128
NCOM = 4
CD = 32

NCOR = 2
NSUB = 16
LANE = 16

KW = 2
WINE = KW * 128
EP = 327680
NP = N + 112
RPS = NP // NSUB


def _sc_mesh():
    return plsc.VectorSubcoreMesh(core_axis_name="c", subcore_axis_name="s")


def _unit_agg_body(tab, ei_r, zeros, out, src_i, dst_i, rows_v, acc_sh, sem):
    c = lax.axis_index("c")
    s = lax.axis_index("s")
    pltpu.sync_copy(zeros.at[pl.ds(s * RPS, RPS)],
                    acc_sh.at[pl.ds(s * RPS, RPS)])
    plsc.subcore_barrier()

    wid = s * NCOR + c
    nw = EP // 32 // WINE

    def window(w, carry):
        row0 = wid * (nw * KW) + w * KW
        pltpu.sync_copy(ei_r.at[0, pl.ds(row0, KW)], src_i)
        pltpu.sync_copy(ei_r.at[1, pl.ds(row0, KW)], dst_i)
        for j in range(KW):
            pltpu.async_copy(tab.at[src_i.at[j]],
                             rows_v.at[pl.ds(j * 128, 128)], sem).wait()
        for j in range(KW):
            pltpu.sync_copy(rows_v.at[pl.ds(j * 128, 128)],
                            acc_sh.at[dst_i.at[j]], add=True)
        return carry

    lax.fori_loop(0, nw, window, 0, unroll=False)
    plsc.subcore_barrier()
    pltpu.sync_copy(acc_sh.at[pl.ds(s * RPS, RPS)],
                    out.at[c, pl.ds(s * RPS, RPS)])


@jax.jit
def _sc_unit_agg(tab, ei_r, zeros):
    return pl.kernel(
        _unit_agg_body,
        out_type=jax.ShapeDtypeStruct((2, NP, 128), jnp.float32),
        mesh=_sc_mesh(),
        scratch_types=[
            pltpu.VMEM((KW, 128), jnp.int32),
            pltpu.VMEM((KW, 128), jnp.int32),
            pltpu.VMEM((WINE, 128), jnp.float32),
            pltpu.VMEM_SHARED((NP, 128), jnp.float32),
            pltpu.SemaphoreType.DMA,
        ],
        compiler_params=pltpu.CompilerParams(use_tc_tiling_on_sc=False),
        name="sc_unit_agg",
    )(tab, ei_r, zeros)


def _make_wagg_body(tw, wine):
    kw = wine // 128

    def body(tab, ei_r, lg, zeros, out, src_i, dst_i, rows_v, msg_v, ew_v,
             acc_sh, sem):
        c = lax.axis_index("c")
        s = lax.axis_index("s")
        pltpu.sync_copy(zeros.at[pl.ds(s * RPS, RPS)],
                        acc_sh.at[pl.ds(s * RPS, RPS)])
        plsc.subcore_barrier()

        wid = s * NCOR + c
        nw = EP // 32 // wine

        def window(w, carry):
            e0 = wid * (nw * wine) + w * wine
            row0 = e0 // 128
            pltpu.sync_copy(ei_r.at[0, pl.ds(row0, kw)], src_i)
            pltpu.sync_copy(ei_r.at[1, pl.ds(row0, kw)], dst_i)
            for j in range(kw):
                pltpu.async_copy(tab.at[src_i.at[j]],
                                 rows_v.at[pl.ds(j * 128, 128)], sem).wait()
            pltpu.sync_copy(lg.at[:, pl.ds(e0, wine)],
                            ew_v.at[:, pl.ds(0, wine)])
            for g in range(wine // 16):
                sl = pl.ds(g * 16, 16)
                l0, l1, l2, l3 = (ew_v[0, sl], ew_v[1, sl], ew_v[2, sl],
                                  ew_v[3, sl])
                m = jnp.maximum(jnp.maximum(l0, l1), jnp.maximum(l2, l3))
                x0, x1 = jnp.exp(l0 - m), jnp.exp(l1 - m)
                x2, x3 = jnp.exp(l2 - m), jnp.exp(l3 - m)
                inv = 1.0 / (x0 + x1 + x2 + x3)
                ew_v[0, sl] = x0 * inv
                ew_v[1, sl] = x1 * inv
                ew_v[2, sl] = x2 * inv
                ew_v[3, sl] = x3 * inv

            def edge(e, carry2):
                if tw == 32:
                    ya = rows_v[e, pl.ds(0, 16)]
                    yb = rows_v[e, pl.ds(16, 16)]
                    for k in range(NCOM):
                        wk = ew_v[k, pl.ds(e, 16)][0]
                        msg_v[e, pl.ds(k * 32, 16)] = ya * wk
                        msg_v[e, pl.ds(k * 32 + 16, 16)] = yb * wk
                else:
                    for k in range(NCOM):
                        wk = ew_v[k, pl.ds(e, 16)][0]
                        a = pl.ds(k * 32, 16)
                        b = pl.ds(k * 32 + 16, 16)
                        msg_v[e, a] = rows_v[e, a] * wk
                        msg_v[e, b] = rows_v[e, b] * wk
                return carry2

            lax.fori_loop(0, wine, edge, 0, unroll=False)
            for j in range(kw):
                pltpu.sync_copy(msg_v.at[pl.ds(j * 128, 128)],
                                acc_sh.at[dst_i.at[j]], add=True)
            return carry

        lax.fori_loop(0, nw, window, 0, unroll=False)
        plsc.subcore_barrier()
        pltpu.sync_copy(acc_sh.at[pl.ds(s * RPS, RPS)],
                        out.at[c, pl.ds(s * RPS, RPS)])

    return body


def _make_wagg(tw, wine):
    kw = wine // 128
    rows_shape = (wine, tw) if tw == 32 else (wine, 128)

    @jax.jit
    def run(tab, ei_r, lg, zeros):
        return pl.kernel(
            _make_wagg_body(tw, wine),
            out_type=jax.ShapeDtypeStruct((2, NP, 128), jnp.float32),
            mesh=_sc_mesh(),
            scratch_types=[
                pltpu.VMEM((kw, 128), jnp.int32),
                pltpu.VMEM((kw, 128), jnp.int32),
                pltpu.VMEM(rows_shape, jnp.float32),
                pltpu.VMEM((wine, 128), jnp.float32),
                pltpu.VMEM((4, wine + 16), jnp.float32),
                pltpu.VMEM_SHARED((NP, 128), jnp.float32),
                pltpu.SemaphoreType.DMA,
            ],
            compiler_params=pltpu.CompilerParams(use_tc_tiling_on_sc=False),
            name=f"sc_wagg{tw}",
        )(tab, ei_r, lg, zeros)

    return run


_sc_wagg32 = _make_wagg(32, 128)
_sc_wagg128 = _make_wagg(128, 128)


WINL = 256
KL = WINL // 128


def _logits_body(z, ei_r, out, src_i, dst_i, zs_v, zd_v, lgo_v, sem):
    c = lax.axis_index("c")
    s = lax.axis_index("s")
    wid = s * NCOR + c
    nw = EP // 32 // WINL
    lanes = jnp.arange(16, dtype=jnp.int32)

    def window(w, carry):
        e0 = wid * (nw * WINL) + w * WINL
        row0 = e0 // 128
        pltpu.sync_copy(ei_r.at[0, pl.ds(row0, KL)], src_i)
        pltpu.sync_copy(ei_r.at[1, pl.ds(row0, KL)], dst_i)
        for j in range(KL):
            pltpu.async_copy(z.at[src_i.at[j]],
                             zs_v.at[pl.ds(j * 128, 128)], sem).wait()
            pltpu.async_copy(z.at[dst_i.at[j]],
                             zd_v.at[pl.ds(j * 128, 128)], sem).wait()

        def gblock(g, carry2):
            rows = g * 16 + lanes
            for k in range(NCOM):
                def dstep(d, acc):
                    cols = jnp.full((16,), d, jnp.int32)
                    vs = plsc.load_gather(zs_v, [rows, cols])
                    vd = plsc.load_gather(zd_v, [rows, cols])
                    return acc + vs * vd

                acc = lax.fori_loop(k * 32, k * 32 + 32, dstep,
                                    jnp.zeros((16,), jnp.float32), unroll=4)
                lgo_v[k, pl.ds(g * 16, 16)] = acc
            return carry2

        lax.fori_loop(0, WINL // 16, gblock, 0, unroll=False)
        pltpu.sync_copy(lgo_v, out.at[:, pl.ds(e0, WINL)])
        return carry

    lax.fori_loop(0, nw, window, 0, unroll=False)


@jax.jit
def _sc_logits(z, ei_r):
    return pl.kernel(
        _logits_body,
        out_type=jax.ShapeDtypeStruct((4, EP), jnp.float32),
        mesh=_sc_mesh(),
        scratch_types=[
            pltpu.VMEM((KL, 128), jnp.int32),
            pltpu.VMEM((KL, 128), jnp.int32),
            pltpu.VMEM((WINL, 128), jnp.float32),
            pltpu.VMEM((WINL, 128), jnp.float32),
            pltpu.VMEM((4, WINL), jnp.float32),
            pltpu.SemaphoreType.DMA,
        ],
        compiler_params=pltpu.CompilerParams(use_tc_tiling_on_sc=False,
                                             needs_layout_passes=False),
        name="sc_logits",
    )(z, ei_r)


def _bn(h, g, b):
    return h * (g / jnp.sqrt(1.0 + 1e-5)) + b


_BNS = float(1.0 / (1.0 + 1e-5) ** 0.5)


def _tc1_body(x_r, W_aff_r, b_aff_r, W0a_r, W_in_r, b_in_r, z_o, y_o, xin_o):
    x = x_r[...]
    z_o[...] = jnp.maximum(
        jnp.dot(x, W_aff_r[...], preferred_element_type=jnp.float32)
        + b_aff_r[...], 0.0)
    y_o[...] = jnp.dot(x, W0a_r[...], preferred_element_type=jnp.float32)
    xin_o[...] = jnp.dot(x, W_in_r[...],
                         preferred_element_type=jnp.float32) + b_in_r[...]


@jax.jit
def _tc1(x, W_aff, b_aff, W0a, W_in, b_in):
    return pl.pallas_call(
        _tc1_body,
        out_shape=[jax.ShapeDtypeStruct((N, D), jnp.float32),
                   jax.ShapeDtypeStruct((N, CD), jnp.float32),
                   jax.ShapeDtypeStruct((N, D), jnp.float32)],
    )(x, W_aff, b_aff.reshape(1, D), W0a, W_in, b_in.reshape(1, D))


def _gin0_body(p_r, y4_r, ba_r, Wb_r, bb_r, bng_r, bnb_r, Wn_r, xin_r,
               hx_o, t_o):
    agg = p_r[0, :N] + p_r[1, :N]
    u = jnp.maximum(y4_r[...] + agg + ba_r[...], 0.0)
    hpre = jnp.dot(u, Wb_r[...], preferred_element_type=jnp.float32) \
        + bb_r[...]
    h = jnp.maximum(hpre * (bng_r[...] * _BNS) + bnb_r[...], 0.0)
    hx_o[...] = h + xin_r[...]
    t_o[...] = jnp.dot(h, Wn_r[...], preferred_element_type=jnp.float32)


@jax.jit
def _gin0(p, y, b0a, W0b, b0b, bn0_g, bn0_b, W1a, xin):
    eye4 = jnp.eye(NCOM, dtype=jnp.float32)
    y4 = jnp.tile(y, (1, NCOM))
    return pl.pallas_call(
        _gin0_body,
        out_shape=[jax.ShapeDtypeStruct((N, D), jnp.float32),
                   jax.ShapeDtypeStruct((N, D), jnp.float32)],
    )(p, y4, jnp.tile(b0a, NCOM).reshape(1, D), jnp.kron(eye4, W0b),
      jnp.tile(b0b, NCOM).reshape(1, D), jnp.tile(bn0_g, NCOM).reshape(1, D),
      jnp.tile(bn0_b, NCOM).reshape(1, D), jnp.kron(eye4, W1a), xin)


def _gin1_body(p_r, t_r, ba_r, Wb_r, bb_r, bng_r, bnb_r, hx_r, Wn_r, hw_o):
    agg = p_r[0, :N] + p_r[1, :N]
    u = jnp.maximum(t_r[...] + agg + ba_r[...], 0.0)
    hpre = jnp.dot(u, Wb_r[...], preferred_element_type=jnp.float32) \
        + bb_r[...]
    h2 = jnp.maximum(hpre * (bng_r[...] * _BNS) + bnb_r[...], 0.0)
    h = hx_r[...] + h2
    hw_o[...] = jnp.dot(h, Wn_r[...], preferred_element_type=jnp.float32)


@jax.jit
def _gin1(p, t, b1a, W1b, b1b, bn1_g, bn1_b, hx, Wc0a):
    eye4 = jnp.eye(NCOM, dtype=jnp.float32)
    return pl.pallas_call(
        _gin1_body,
        out_shape=jax.ShapeDtypeStruct((N, D), jnp.float32),
    )(p, t, jnp.tile(b1a, NCOM).reshape(1, D), jnp.kron(eye4, W1b),
      jnp.tile(b1b, NCOM).reshape(1, D), jnp.tile(bn1_g, NCOM).reshape(1, D),
      jnp.tile(bn1_b, NCOM).reshape(1, D), hx, Wc0a)


def _rep_mid_body(p_r, base_r, ba_r, Wb_r, bb_r, bng_r, bnb_r, Wn_r, gw_o):
    agg = p_r[0, :N] + p_r[1, :N]
    u = jnp.maximum(base_r[...] + agg + ba_r[...], 0.0)
    hpre = jnp.dot(u, Wb_r[...], preferred_element_type=jnp.float32) \
        + bb_r[...]
    g1 = jnp.maximum(hpre * (bng_r[...] * _BNS) + bnb_r[...], 0.0)
    gw_o[...] = jnp.dot(g1, Wn_r[...], preferred_element_type=jnp.float32)


@jax.jit
def _rep_mid(p, base, ba, Wb, bb, bng, bnb, Wn):
    return pl.pallas_call(
        _rep_mid_body,
        out_shape=jax.ShapeDtypeStruct((N, D), jnp.float32),
    )(p, base, ba.reshape(1, D), Wb, bb.reshape(1, D), bng.reshape(1, D),
      bnb.reshape(1, D), Wn)


def _rep_final_body(p_r, base_r, ba_r, Wb_r, bb_r, bng_r, bnb_r, g2_o):
    agg = p_r[0, :N] + p_r[1, :N]
    u = jnp.maximum(base_r[...] + agg + ba_r[...], 0.0)
    hpre = jnp.dot(u, Wb_r[...], preferred_element_type=jnp.float32) \
        + bb_r[...]
    g2_o[...] = jnp.maximum(hpre * (bng_r[...] * _BNS) + bnb_r[...], 0.0)


@jax.jit
def _rep_final(p, base, ba, Wb, bb, bng, bnb):
    return pl.pallas_call(
        _rep_final_body,
        out_shape=jax.ShapeDtypeStruct((N, D), jnp.float32),
    )(p, base, ba.reshape(1, D), Wb, bb.reshape(1, D), bng.reshape(1, D),
      bnb.reshape(1, D))


def kernel(x, edge_index, W_aff, b_aff, W0a, b0a, W0b, b0b, bn0_g, bn0_b,
           W1a, b1a, W1b, b1b, bn1_g, bn1_b, W_in, b_in,
           Wc0a, bc0a, Wc0b, bc0b, bnc0_g, bnc0_b,
           Wc1a, bc1a, Wc1b, bc1b, bnc1_g, bnc1_b):
    src, dst = edge_index[0], edge_index[1]

    npad = EP - E
    pad_lane = (jnp.arange(npad, dtype=jnp.int32) % 112)
    ei_r = jnp.concatenate([
        jnp.stack([src, dst]),
        jnp.stack([pad_lane, N + pad_lane]),
    ], axis=1).reshape(2, EP // 128, 128)
    zerosNP = jnp.zeros((NP, 128), jnp.float32)

    z, y, xin = _tc1(x, W_aff, b_aff, W0a, W_in, b_in)

    lg = _sc_logits(z, ei_r)

    pB = _sc_wagg32(y, ei_r, lg, zerosNP)
    hx, t = _gin0(pB, y, b0a, W0b, b0b, bn0_g, bn0_b, W1a, xin)

    pC = _sc_wagg128(t, ei_r, lg, zerosNP)
    hw = _gin1(pC, t, b1a, W1b, b1b, bn1_g, bn1_b, hx, Wc0a)

    p1 = _sc_unit_agg(hw, ei_r, zerosNP)
    gw = _rep_mid(p1, hw, bc0a, Wc0b, bc0b, bnc0_g, bnc0_b, Wc1a)

    p2 = _sc_unit_agg(gw, ei_r, zerosNP)
    g2 = _rep_final(p2, gw, bc1a, Wc1b, bc1b, bnc1_g, bnc1_b)
    return g2


# speedup vs baseline: 4.7129x; 1.0256x over previous
import functools

import jax
import jax.numpy as jnp
from jax import lax
from jax.experimental import pallas as pl
from jax.experimental.pallas import tpu as pltpu
from jax.experimental.pallas import tpu_sc as plsc

N = 10000
E = 320000
D = 128
NCOM = 4
CD = 32

NCOR = 2
NSUB = 16
LANE = 16

KW = 2
WINE = KW * 128
EP = 327680
NP = N + 112
RPS = NP // NSUB


def _sc_mesh():
    return plsc.VectorSubcoreMesh(core_axis_name="c", subcore_axis_name="s")


def _unit_agg_body(tab, ei_r, zeros, out, src_i, dst_i, rows_v, acc_sh, sem):
    c = lax.axis_index("c")
    s = lax.axis_index("s")
    pltpu.sync_copy(zeros.at[pl.ds(s * RPS, RPS)],
                    acc_sh.at[pl.ds(s * RPS, RPS)])
    plsc.subcore_barrier()

    wid = s * NCOR + c
    nw = EP // 32 // WINE

    def window(w, carry):
        row0 = wid * (nw * KW) + w * KW
        pltpu.sync_copy(ei_r.at[0, pl.ds(row0, KW)], src_i)
        pltpu.sync_copy(ei_r.at[1, pl.ds(row0, KW)], dst_i)
        for j in range(KW):
            pltpu.async_copy(tab.at[src_i.at[j]],
                             rows_v.at[pl.ds(j * 128, 128)], sem).wait()
        for j in range(KW):
            pltpu.sync_copy(rows_v.at[pl.ds(j * 128, 128)],
                            acc_sh.at[dst_i.at[j]], add=True)
        return carry

    lax.fori_loop(0, nw, window, 0, unroll=False)
    plsc.subcore_barrier()
    pltpu.sync_copy(acc_sh.at[pl.ds(s * RPS, RPS)],
                    out.at[c, pl.ds(s * RPS, RPS)])


@jax.jit
def _sc_unit_agg(tab, ei_r, zeros):
    return pl.kernel(
        _unit_agg_body,
        out_type=jax.ShapeDtypeStruct((2, NP, 128), jnp.float32),
        mesh=_sc_mesh(),
        scratch_types=[
            pltpu.VMEM((KW, 128), jnp.int32),
            pltpu.VMEM((KW, 128), jnp.int32),
            pltpu.VMEM((WINE, 128), jnp.float32),
            pltpu.VMEM_SHARED((NP, 128), jnp.float32),
            pltpu.SemaphoreType.DMA,
        ],
        compiler_params=pltpu.CompilerParams(use_tc_tiling_on_sc=False),
        name="sc_unit_agg",
    )(tab, ei_r, zeros)


def _make_wagg_body(tw, wine):
    kw = wine // 128

    def body(tab, ei_r, lg, zeros, out, src_i, dst_i, rows_v, msg_v, ew_v,
             acc_sh, sem):
        c = lax.axis_index("c")
        s = lax.axis_index("s")
        pltpu.sync_copy(zeros.at[pl.ds(s * RPS, RPS)],
                        acc_sh.at[pl.ds(s * RPS, RPS)])
        plsc.subcore_barrier()

        wid = s * NCOR + c
        nw = EP // 32 // wine

        def window(w, carry):
            e0 = wid * (nw * wine) + w * wine
            row0 = e0 // 128
            pltpu.sync_copy(ei_r.at[0, pl.ds(row0, kw)], src_i)
            pltpu.sync_copy(ei_r.at[1, pl.ds(row0, kw)], dst_i)
            for j in range(kw):
                pltpu.async_copy(tab.at[src_i.at[j]],
                                 rows_v.at[pl.ds(j * 128, 128)], sem).wait()
            pltpu.sync_copy(lg.at[:, pl.ds(e0, wine)],
                            ew_v.at[:, pl.ds(0, wine)])
            for g in range(wine // 16):
                sl = pl.ds(g * 16, 16)
                l0, l1, l2, l3 = (ew_v[0, sl], ew_v[1, sl], ew_v[2, sl],
                                  ew_v[3, sl])
                m = jnp.maximum(jnp.maximum(l0, l1), jnp.maximum(l2, l3))
                x0, x1 = jnp.exp(l0 - m), jnp.exp(l1 - m)
                x2, x3 = jnp.exp(l2 - m), jnp.exp(l3 - m)
                inv = 1.0 / (x0 + x1 + x2 + x3)
                ew_v[0, sl] = x0 * inv
                ew_v[1, sl] = x1 * inv
                ew_v[2, sl] = x2 * inv
                ew_v[3, sl] = x3 * inv

            def edge(e, carry2):
                if tw == 32:
                    ya = rows_v[e, pl.ds(0, 16)]
                    yb = rows_v[e, pl.ds(16, 16)]
                    for k in range(NCOM):
                        wk = ew_v[k, pl.ds(e, 16)][0]
                        msg_v[e, pl.ds(k * 32, 16)] = ya * wk
                        msg_v[e, pl.ds(k * 32 + 16, 16)] = yb * wk
                else:
                    for k in range(NCOM):
                        wk = ew_v[k, pl.ds(e, 16)][0]
                        a = pl.ds(k * 32, 16)
                        b = pl.ds(k * 32 + 16, 16)
                        msg_v[e, a] = rows_v[e, a] * wk
                        msg_v[e, b] = rows_v[e, b] * wk
                return carry2

            lax.fori_loop(0, wine, edge, 0, unroll=False)
            for j in range(kw):
                pltpu.sync_copy(msg_v.at[pl.ds(j * 128, 128)],
                                acc_sh.at[dst_i.at[j]], add=True)
            return carry

        lax.fori_loop(0, nw, window, 0, unroll=False)
        plsc.subcore_barrier()
        pltpu.sync_copy(acc_sh.at[pl.ds(s * RPS, RPS)],
                        out.at[c, pl.ds(s * RPS, RPS)])

    return body


def _make_wagg(tw, wine):
    kw = wine // 128
    rows_shape = (wine, tw) if tw == 32 else (wine, 128)

    @jax.jit
    def run(tab, ei_r, lg, zeros):
        return pl.kernel(
            _make_wagg_body(tw, wine),
            out_type=jax.ShapeDtypeStruct((2, NP, 128), jnp.float32),
            mesh=_sc_mesh(),
            scratch_types=[
                pltpu.VMEM((kw, 128), jnp.int32),
                pltpu.VMEM((kw, 128), jnp.int32),
                pltpu.VMEM(rows_shape, jnp.float32),
                pltpu.VMEM((wine, 128), jnp.float32),
                pltpu.VMEM((4, wine + 16), jnp.float32),
                pltpu.VMEM_SHARED((NP, 128), jnp.float32),
                pltpu.SemaphoreType.DMA,
            ],
            compiler_params=pltpu.CompilerParams(use_tc_tiling_on_sc=False),
            name=f"sc_wagg{tw}",
        )(tab, ei_r, lg, zeros)

    return run


_sc_wagg32 = _make_wagg(32, 128)
_sc_wagg128 = _make_wagg(128, 128)


WINL = 256
KL = WINL // 128


def _logits_body(z, ei_r, out, src_i, dst_i, zs_v, zd_v, lgo_v, sem):
    c = lax.axis_index("c")
    s = lax.axis_index("s")
    wid = s * NCOR + c
    nw = EP // 32 // WINL
    lanes = jnp.arange(16, dtype=jnp.int32)

    def window(w, carry):
        e0 = wid * (nw * WINL) + w * WINL
        row0 = e0 // 128
        pltpu.sync_copy(ei_r.at[0, pl.ds(row0, KL)], src_i)
        pltpu.sync_copy(ei_r.at[1, pl.ds(row0, KL)], dst_i)
        descs = []
        for j in range(KL):
            descs.append(pltpu.async_copy(
                z.at[src_i.at[j]], zs_v.at[pl.ds(j * 128, 128)], sem))
            descs.append(pltpu.async_copy(
                z.at[dst_i.at[j]], zd_v.at[pl.ds(j * 128, 128)], sem))
        for dsc in descs:
            dsc.wait()

        def gblock(g, carry2):
            rows = g * 16 + lanes
            for k in range(NCOM):
                def dstep(d, acc):
                    cols = jnp.full((16,), d, jnp.int32)
                    vs = plsc.load_gather(zs_v, [rows, cols])
                    vd = plsc.load_gather(zd_v, [rows, cols])
                    return acc + vs * vd

                acc = lax.fori_loop(k * 32, k * 32 + 32, dstep,
                                    jnp.zeros((16,), jnp.float32), unroll=4)
                lgo_v[k, pl.ds(g * 16, 16)] = acc
            return carry2

        lax.fori_loop(0, WINL // 16, gblock, 0, unroll=False)
        pltpu.sync_copy(lgo_v, out.at[:, pl.ds(e0, WINL)])
        return carry

    lax.fori_loop(0, nw, window, 0, unroll=False)


@jax.jit
def _sc_logits(z, ei_r):
    return pl.kernel(
        _logits_body,
        out_type=jax.ShapeDtypeStruct((4, EP), jnp.float32),
        mesh=_sc_mesh(),
        scratch_types=[
            pltpu.VMEM((KL, 128), jnp.int32),
            pltpu.VMEM((KL, 128), jnp.int32),
            pltpu.VMEM((WINL, 128), jnp.float32),
            pltpu.VMEM((WINL, 128), jnp.float32),
            pltpu.VMEM((4, WINL), jnp.float32),
            pltpu.SemaphoreType.DMA,
        ],
        compiler_params=pltpu.CompilerParams(use_tc_tiling_on_sc=False,
                                             needs_layout_passes=False),
        name="sc_logits",
    )(z, ei_r)


def _bn(h, g, b):
    return h * (g / jnp.sqrt(1.0 + 1e-5)) + b


_BNS = float(1.0 / (1.0 + 1e-5) ** 0.5)


def _tc1_body(x_r, W_aff_r, b_aff_r, W0a_r, W_in_r, b_in_r, z_o, y_o, xin_o):
    x = x_r[...]
    z_o[...] = jnp.maximum(
        jnp.dot(x, W_aff_r[...], preferred_element_type=jnp.float32)
        + b_aff_r[...], 0.0)
    y_o[...] = jnp.dot(x, W0a_r[...], preferred_element_type=jnp.float32)
    xin_o[...] = jnp.dot(x, W_in_r[...],
                         preferred_element_type=jnp.float32) + b_in_r[...]


@jax.jit
def _tc1(x, W_aff, b_aff, W0a, W_in, b_in):
    return pl.pallas_call(
        _tc1_body,
        out_shape=[jax.ShapeDtypeStruct((N, D), jnp.float32),
                   jax.ShapeDtypeStruct((N, CD), jnp.float32),
                   jax.ShapeDtypeStruct((N, D), jnp.float32)],
    )(x, W_aff, b_aff.reshape(1, D), W0a, W_in, b_in.reshape(1, D))


def _gin0_body(p_r, y4_r, ba_r, Wb_r, bb_r, bng_r, bnb_r, Wn_r, xin_r,
               hx_o, t_o):
    agg = p_r[0, :N] + p_r[1, :N]
    u = jnp.maximum(y4_r[...] + agg + ba_r[...], 0.0)
    hpre = jnp.dot(u, Wb_r[...], preferred_element_type=jnp.float32) \
        + bb_r[...]
    h = jnp.maximum(hpre * (bng_r[...] * _BNS) + bnb_r[...], 0.0)
    hx_o[...] = h + xin_r[...]
    t_o[...] = jnp.dot(h, Wn_r[...], preferred_element_type=jnp.float32)


@jax.jit
def _gin0(p, y, b0a, W0b, b0b, bn0_g, bn0_b, W1a, xin):
    eye4 = jnp.eye(NCOM, dtype=jnp.float32)
    y4 = jnp.tile(y, (1, NCOM))
    return pl.pallas_call(
        _gin0_body,
        out_shape=[jax.ShapeDtypeStruct((N, D), jnp.float32),
                   jax.ShapeDtypeStruct((N, D), jnp.float32)],
    )(p, y4, jnp.tile(b0a, NCOM).reshape(1, D), jnp.kron(eye4, W0b),
      jnp.tile(b0b, NCOM).reshape(1, D), jnp.tile(bn0_g, NCOM).reshape(1, D),
      jnp.tile(bn0_b, NCOM).reshape(1, D), jnp.kron(eye4, W1a), xin)


def _gin1_body(p_r, t_r, ba_r, Wb_r, bb_r, bng_r, bnb_r, hx_r, Wn_r, hw_o):
    agg = p_r[0, :N] + p_r[1, :N]
    u = jnp.maximum(t_r[...] + agg + ba_r[...], 0.0)
    hpre = jnp.dot(u, Wb_r[...], preferred_element_type=jnp.float32) \
        + bb_r[...]
    h2 = jnp.maximum(hpre * (bng_r[...] * _BNS) + bnb_r[...], 0.0)
    h = hx_r[...] + h2
    hw_o[...] = jnp.dot(h, Wn_r[...], preferred_element_type=jnp.float32)


@jax.jit
def _gin1(p, t, b1a, W1b, b1b, bn1_g, bn1_b, hx, Wc0a):
    eye4 = jnp.eye(NCOM, dtype=jnp.float32)
    return pl.pallas_call(
        _gin1_body,
        out_shape=jax.ShapeDtypeStruct((N, D), jnp.float32),
    )(p, t, jnp.tile(b1a, NCOM).reshape(1, D), jnp.kron(eye4, W1b),
      jnp.tile(b1b, NCOM).reshape(1, D), jnp.tile(bn1_g, NCOM).reshape(1, D),
      jnp.tile(bn1_b, NCOM).reshape(1, D), hx, Wc0a)


def _rep_mid_body(p_r, base_r, ba_r, Wb_r, bb_r, bng_r, bnb_r, Wn_r, gw_o):
    agg = p_r[0, :N] + p_r[1, :N]
    u = jnp.maximum(base_r[...] + agg + ba_r[...], 0.0)
    hpre = jnp.dot(u, Wb_r[...], preferred_element_type=jnp.float32) \
        + bb_r[...]
    g1 = jnp.maximum(hpre * (bng_r[...] * _BNS) + bnb_r[...], 0.0)
    gw_o[...] = jnp.dot(g1, Wn_r[...], preferred_element_type=jnp.float32)


@jax.jit
def _rep_mid(p, base, ba, Wb, bb, bng, bnb, Wn):
    return pl.pallas_call(
        _rep_mid_body,
        out_shape=jax.ShapeDtypeStruct((N, D), jnp.float32),
    )(p, base, ba.reshape(1, D), Wb, bb.reshape(1, D), bng.reshape(1, D),
      bnb.reshape(1, D), Wn)


def _rep_final_body(p_r, base_r, ba_r, Wb_r, bb_r, bng_r, bnb_r, g2_o):
    agg = p_r[0, :N] + p_r[1, :N]
    u = jnp.maximum(base_r[...] + agg + ba_r[...], 0.0)
    hpre = jnp.dot(u, Wb_r[...], preferred_element_type=jnp.float32) \
        + bb_r[...]
    g2_o[...] = jnp.maximum(hpre * (bng_r[...] * _BNS) + bnb_r[...], 0.0)


@jax.jit
def _rep_final(p, base, ba, Wb, bb, bng, bnb):
    return pl.pallas_call(
        _rep_final_body,
        out_shape=jax.ShapeDtypeStruct((N, D), jnp.float32),
    )(p, base, ba.reshape(1, D), Wb, bb.reshape(1, D), bng.reshape(1, D),
      bnb.reshape(1, D))


def kernel(x, edge_index, W_aff, b_aff, W0a, b0a, W0b, b0b, bn0_g, bn0_b,
           W1a, b1a, W1b, b1b, bn1_g, bn1_b, W_in, b_in,
           Wc0a, bc0a, Wc0b, bc0b, bnc0_g, bnc0_b,
           Wc1a, bc1a, Wc1b, bc1b, bnc1_g, bnc1_b):
    src, dst = edge_index[0], edge_index[1]

    npad = EP - E
    pad_lane = (jnp.arange(npad, dtype=jnp.int32) % 112)
    ei_r = jnp.concatenate([
        jnp.stack([src, dst]),
        jnp.stack([pad_lane, N + pad_lane]),
    ], axis=1).reshape(2, EP // 128, 128)
    zerosNP = jnp.zeros((NP, 128), jnp.float32)

    z, y, xin = _tc1(x, W_aff, b_aff, W0a, W_in, b_in)

    lg = _sc_logits(z, ei_r)

    pB = _sc_wagg32(y, ei_r, lg, zerosNP)
    hx, t = _gin0(pB, y, b0a, W0b, b0b, bn0_g, bn0_b, W1a, xin)

    pC = _sc_wagg128(t, ei_r, lg, zerosNP)
    hw = _gin1(pC, t, b1a, W1b, b1b, bn1_g, bn1_b, hx, Wc0a)

    p1 = _sc_unit_agg(hw, ei_r, zerosNP)
    gw = _rep_mid(p1, hw, bc0a, Wc0b, bc0b, bnc0_g, bnc0_b, Wc1a)

    p2 = _sc_unit_agg(gw, ei_r, zerosNP)
    g2 = _rep_final(p2, gw, bc1a, Wc1b, bc1b, bnc1_g, bnc1_b)
    return g2


# speedup vs baseline: 7.0993x; 1.5063x over previous
import functools

import jax
import jax.numpy as jnp
from jax import lax
from jax.experimental import pallas as pl
from jax.experimental.pallas import tpu as pltpu
from jax.experimental.pallas import tpu_sc as plsc

N = 10000
E = 320000
D = 128
NCOM = 4
CD = 32

NCOR = 2
NSUB = 16
LANE = 16

KW = 2
WINE = KW * 128
EP = 327680
NP = N + 112
RPS = NP // NSUB


def _sc_mesh():
    return plsc.VectorSubcoreMesh(core_axis_name="c", subcore_axis_name="s")


def _unit_agg_body(tab, ei_r, zeros, out, src_i, dst_i, rows_v, acc_sh, sem):
    c = lax.axis_index("c")
    s = lax.axis_index("s")
    pltpu.sync_copy(zeros.at[pl.ds(s * RPS, RPS)],
                    acc_sh.at[pl.ds(s * RPS, RPS)])
    plsc.subcore_barrier()

    wid = s * NCOR + c
    nw = EP // 32 // WINE

    def window(w, carry):
        row0 = wid * (nw * KW) + w * KW
        pltpu.sync_copy(ei_r.at[0, pl.ds(row0, KW)], src_i)
        pltpu.sync_copy(ei_r.at[1, pl.ds(row0, KW)], dst_i)
        for j in range(KW):
            pltpu.async_copy(tab.at[src_i.at[j]],
                             rows_v.at[pl.ds(j * 128, 128)], sem).wait()
        for j in range(KW):
            pltpu.sync_copy(rows_v.at[pl.ds(j * 128, 128)],
                            acc_sh.at[dst_i.at[j]], add=True)
        return carry

    lax.fori_loop(0, nw, window, 0, unroll=False)
    plsc.subcore_barrier()
    pltpu.sync_copy(acc_sh.at[pl.ds(s * RPS, RPS)],
                    out.at[c, pl.ds(s * RPS, RPS)])


@jax.jit
def _sc_unit_agg(tab, ei_r, zeros):
    return pl.kernel(
        _unit_agg_body,
        out_type=jax.ShapeDtypeStruct((2, NP, 128), jnp.float32),
        mesh=_sc_mesh(),
        scratch_types=[
            pltpu.VMEM((KW, 128), jnp.int32),
            pltpu.VMEM((KW, 128), jnp.int32),
            pltpu.VMEM((WINE, 128), jnp.float32),
            pltpu.VMEM_SHARED((NP, 128), jnp.float32),
            pltpu.SemaphoreType.DMA,
        ],
        compiler_params=pltpu.CompilerParams(use_tc_tiling_on_sc=False),
        name="sc_unit_agg",
    )(tab, ei_r, zeros)


def _make_wagg_body(tw, wine):
    kw = wine // 128

    def body(tab, ei_r, lg, zeros, out, src_i, dst_i, rows_v, msg_v, ew_v,
             acc_sh, sem):
        c = lax.axis_index("c")
        s = lax.axis_index("s")
        pltpu.sync_copy(zeros.at[pl.ds(s * RPS, RPS)],
                        acc_sh.at[pl.ds(s * RPS, RPS)])
        plsc.subcore_barrier()

        wid = s * NCOR + c
        nw = EP // 32 // wine

        def window(w, carry):
            e0 = wid * (nw * wine) + w * wine
            row0 = e0 // 128
            pltpu.sync_copy(ei_r.at[0, pl.ds(row0, kw)], src_i)
            pltpu.sync_copy(ei_r.at[1, pl.ds(row0, kw)], dst_i)
            for j in range(kw):
                pltpu.async_copy(tab.at[src_i.at[j]],
                                 rows_v.at[pl.ds(j * 128, 128)], sem).wait()
            pltpu.sync_copy(lg.at[:, pl.ds(e0, wine)],
                            ew_v.at[:, pl.ds(0, wine)])
            for g in range(wine // 16):
                sl = pl.ds(g * 16, 16)
                l0, l1, l2, l3 = (ew_v[0, sl], ew_v[1, sl], ew_v[2, sl],
                                  ew_v[3, sl])
                m = jnp.maximum(jnp.maximum(l0, l1), jnp.maximum(l2, l3))
                x0, x1 = jnp.exp(l0 - m), jnp.exp(l1 - m)
                x2, x3 = jnp.exp(l2 - m), jnp.exp(l3 - m)
                inv = 1.0 / (x0 + x1 + x2 + x3)
                ew_v[0, sl] = x0 * inv
                ew_v[1, sl] = x1 * inv
                ew_v[2, sl] = x2 * inv
                ew_v[3, sl] = x3 * inv

            def edge(e, carry2):
                if tw == 32:
                    ya = rows_v[e, pl.ds(0, 16)]
                    yb = rows_v[e, pl.ds(16, 16)]
                    for k in range(NCOM):
                        wk = ew_v[k, pl.ds(e, 16)][0]
                        msg_v[e, pl.ds(k * 32, 16)] = ya * wk
                        msg_v[e, pl.ds(k * 32 + 16, 16)] = yb * wk
                else:
                    for k in range(NCOM):
                        wk = ew_v[k, pl.ds(e, 16)][0]
                        a = pl.ds(k * 32, 16)
                        b = pl.ds(k * 32 + 16, 16)
                        msg_v[e, a] = rows_v[e, a] * wk
                        msg_v[e, b] = rows_v[e, b] * wk
                return carry2

            lax.fori_loop(0, wine, edge, 0, unroll=False)
            for j in range(kw):
                pltpu.sync_copy(msg_v.at[pl.ds(j * 128, 128)],
                                acc_sh.at[dst_i.at[j]], add=True)
            return carry

        lax.fori_loop(0, nw, window, 0, unroll=False)
        plsc.subcore_barrier()
        pltpu.sync_copy(acc_sh.at[pl.ds(s * RPS, RPS)],
                        out.at[c, pl.ds(s * RPS, RPS)])

    return body


def _make_wagg(tw, wine):
    kw = wine // 128
    rows_shape = (wine, tw) if tw == 32 else (wine, 128)

    @jax.jit
    def run(tab, ei_r, lg, zeros):
        return pl.kernel(
            _make_wagg_body(tw, wine),
            out_type=jax.ShapeDtypeStruct((2, NP, 128), jnp.float32),
            mesh=_sc_mesh(),
            scratch_types=[
                pltpu.VMEM((kw, 128), jnp.int32),
                pltpu.VMEM((kw, 128), jnp.int32),
                pltpu.VMEM(rows_shape, jnp.float32),
                pltpu.VMEM((wine, 128), jnp.float32),
                pltpu.VMEM((4, wine + 16), jnp.float32),
                pltpu.VMEM_SHARED((NP, 128), jnp.float32),
                pltpu.SemaphoreType.DMA,
            ],
            compiler_params=pltpu.CompilerParams(use_tc_tiling_on_sc=False),
            name=f"sc_wagg{tw}",
        )(tab, ei_r, lg, zeros)

    return run


_sc_wagg32 = _make_wagg(32, 128)
_sc_wagg128 = _make_wagg(128, 128)


WINL = 256
KL = WINL // 128


def _logits_body(z, ei_r, out, src_i, dst_i, zs_v, zd_v, lgo_v, sem):
    c = lax.axis_index("c")
    s = lax.axis_index("s")
    wid = s * NCOR + c
    nw = EP // 32 // WINL
    lanes = jnp.arange(16, dtype=jnp.int32)

    def window(w, carry):
        e0 = wid * (nw * WINL) + w * WINL
        row0 = e0 // 128
        pltpu.sync_copy(ei_r.at[0, pl.ds(row0, KL)], src_i)
        pltpu.sync_copy(ei_r.at[1, pl.ds(row0, KL)], dst_i)
        descs = []
        for j in range(KL):
            descs.append(pltpu.async_copy(
                z.at[src_i.at[j]], zs_v.at[pl.ds(j * 128, 128)], sem))
            descs.append(pltpu.async_copy(
                z.at[dst_i.at[j]], zd_v.at[pl.ds(j * 128, 128)], sem))
        for dsc in descs:
            dsc.wait()

        def gblock(g, carry2):
            acc = [jnp.zeros((16,), jnp.float32) for _ in range(NCOM)]
            for j in range(16):
                e = g * 16 + j
                for k in range(NCOM):
                    a = zs_v[e, pl.ds(k * 32, 16)] * zd_v[e, pl.ds(k * 32, 16)]
                    b = (zs_v[e, pl.ds(k * 32 + 16, 16)]
                         * zd_v[e, pl.ds(k * 32 + 16, 16)])
                    sk = jnp.sum(a + b)
                    acc[k] = jnp.where(lanes == j, sk, acc[k])
            for k in range(NCOM):
                lgo_v[k, pl.ds(g * 16, 16)] = acc[k]
            return carry2

        lax.fori_loop(0, WINL // 16, gblock, 0, unroll=False)
        pltpu.sync_copy(lgo_v, out.at[:, pl.ds(e0, WINL)])
        return carry

    lax.fori_loop(0, nw, window, 0, unroll=False)


@jax.jit
def _sc_logits(z, ei_r):
    return pl.kernel(
        _logits_body,
        out_type=jax.ShapeDtypeStruct((4, EP), jnp.float32),
        mesh=_sc_mesh(),
        scratch_types=[
            pltpu.VMEM((KL, 128), jnp.int32),
            pltpu.VMEM((KL, 128), jnp.int32),
            pltpu.VMEM((WINL, 128), jnp.float32),
            pltpu.VMEM((WINL, 128), jnp.float32),
            pltpu.VMEM((4, WINL), jnp.float32),
            pltpu.SemaphoreType.DMA,
        ],
        compiler_params=pltpu.CompilerParams(use_tc_tiling_on_sc=False,
                                             needs_layout_passes=False),
        name="sc_logits",
    )(z, ei_r)


def _bn(h, g, b):
    return h * (g / jnp.sqrt(1.0 + 1e-5)) + b


_BNS = float(1.0 / (1.0 + 1e-5) ** 0.5)


def _tc1_body(x_r, W_aff_r, b_aff_r, W0a_r, W_in_r, b_in_r, z_o, y_o, xin_o):
    x = x_r[...]
    z_o[...] = jnp.maximum(
        jnp.dot(x, W_aff_r[...], preferred_element_type=jnp.float32)
        + b_aff_r[...], 0.0)
    y_o[...] = jnp.dot(x, W0a_r[...], preferred_element_type=jnp.float32)
    xin_o[...] = jnp.dot(x, W_in_r[...],
                         preferred_element_type=jnp.float32) + b_in_r[...]


@jax.jit
def _tc1(x, W_aff, b_aff, W0a, W_in, b_in):
    return pl.pallas_call(
        _tc1_body,
        out_shape=[jax.ShapeDtypeStruct((N, D), jnp.float32),
                   jax.ShapeDtypeStruct((N, CD), jnp.float32),
                   jax.ShapeDtypeStruct((N, D), jnp.float32)],
    )(x, W_aff, b_aff.reshape(1, D), W0a, W_in, b_in.reshape(1, D))


def _gin0_body(p_r, y4_r, ba_r, Wb_r, bb_r, bng_r, bnb_r, Wn_r, xin_r,
               hx_o, t_o):
    agg = p_r[0, :N] + p_r[1, :N]
    u = jnp.maximum(y4_r[...] + agg + ba_r[...], 0.0)
    hpre = jnp.dot(u, Wb_r[...], preferred_element_type=jnp.float32) \
        + bb_r[...]
    h = jnp.maximum(hpre * (bng_r[...] * _BNS) + bnb_r[...], 0.0)
    hx_o[...] = h + xin_r[...]
    t_o[...] = jnp.dot(h, Wn_r[...], preferred_element_type=jnp.float32)


@jax.jit
def _gin0(p, y, b0a, W0b, b0b, bn0_g, bn0_b, W1a, xin):
    eye4 = jnp.eye(NCOM, dtype=jnp.float32)
    y4 = jnp.tile(y, (1, NCOM))
    return pl.pallas_call(
        _gin0_body,
        out_shape=[jax.ShapeDtypeStruct((N, D), jnp.float32),
                   jax.ShapeDtypeStruct((N, D), jnp.float32)],
    )(p, y4, jnp.tile(b0a, NCOM).reshape(1, D), jnp.kron(eye4, W0b),
      jnp.tile(b0b, NCOM).reshape(1, D), jnp.tile(bn0_g, NCOM).reshape(1, D),
      jnp.tile(bn0_b, NCOM).reshape(1, D), jnp.kron(eye4, W1a), xin)


def _gin1_body(p_r, t_r, ba_r, Wb_r, bb_r, bng_r, bnb_r, hx_r, Wn_r, hw_o):
    agg = p_r[0, :N] + p_r[1, :N]
    u = jnp.maximum(t_r[...] + agg + ba_r[...], 0.0)
    hpre = jnp.dot(u, Wb_r[...], preferred_element_type=jnp.float32) \
        + bb_r[...]
    h2 = jnp.maximum(hpre * (bng_r[...] * _BNS) + bnb_r[...], 0.0)
    h = hx_r[...] + h2
    hw_o[...] = jnp.dot(h, Wn_r[...], preferred_element_type=jnp.float32)


@jax.jit
def _gin1(p, t, b1a, W1b, b1b, bn1_g, bn1_b, hx, Wc0a):
    eye4 = jnp.eye(NCOM, dtype=jnp.float32)
    return pl.pallas_call(
        _gin1_body,
        out_shape=jax.ShapeDtypeStruct((N, D), jnp.float32),
    )(p, t, jnp.tile(b1a, NCOM).reshape(1, D), jnp.kron(eye4, W1b),
      jnp.tile(b1b, NCOM).reshape(1, D), jnp.tile(bn1_g, NCOM).reshape(1, D),
      jnp.tile(bn1_b, NCOM).reshape(1, D), hx, Wc0a)


def _rep_mid_body(p_r, base_r, ba_r, Wb_r, bb_r, bng_r, bnb_r, Wn_r, gw_o):
    agg = p_r[0, :N] + p_r[1, :N]
    u = jnp.maximum(base_r[...] + agg + ba_r[...], 0.0)
    hpre = jnp.dot(u, Wb_r[...], preferred_element_type=jnp.float32) \
        + bb_r[...]
    g1 = jnp.maximum(hpre * (bng_r[...] * _BNS) + bnb_r[...], 0.0)
    gw_o[...] = jnp.dot(g1, Wn_r[...], preferred_element_type=jnp.float32)


@jax.jit
def _rep_mid(p, base, ba, Wb, bb, bng, bnb, Wn):
    return pl.pallas_call(
        _rep_mid_body,
        out_shape=jax.ShapeDtypeStruct((N, D), jnp.float32),
    )(p, base, ba.reshape(1, D), Wb, bb.reshape(1, D), bng.reshape(1, D),
      bnb.reshape(1, D), Wn)


def _rep_final_body(p_r, base_r, ba_r, Wb_r, bb_r, bng_r, bnb_r, g2_o):
    agg = p_r[0, :N] + p_r[1, :N]
    u = jnp.maximum(base_r[...] + agg + ba_r[...], 0.0)
    hpre = jnp.dot(u, Wb_r[...], preferred_element_type=jnp.float32) \
        + bb_r[...]
    g2_o[...] = jnp.maximum(hpre * (bng_r[...] * _BNS) + bnb_r[...], 0.0)


@jax.jit
def _rep_final(p, base, ba, Wb, bb, bng, bnb):
    return pl.pallas_call(
        _rep_final_body,
        out_shape=jax.ShapeDtypeStruct((N, D), jnp.float32),
    )(p, base, ba.reshape(1, D), Wb, bb.reshape(1, D), bng.reshape(1, D),
      bnb.reshape(1, D))


def kernel(x, edge_index, W_aff, b_aff, W0a, b0a, W0b, b0b, bn0_g, bn0_b,
           W1a, b1a, W1b, b1b, bn1_g, bn1_b, W_in, b_in,
           Wc0a, bc0a, Wc0b, bc0b, bnc0_g, bnc0_b,
           Wc1a, bc1a, Wc1b, bc1b, bnc1_g, bnc1_b):
    src, dst = edge_index[0], edge_index[1]

    npad = EP - E
    pad_lane = (jnp.arange(npad, dtype=jnp.int32) % 112)
    ei_r = jnp.concatenate([
        jnp.stack([src, dst]),
        jnp.stack([pad_lane, N + pad_lane]),
    ], axis=1).reshape(2, EP // 128, 128)
    zerosNP = jnp.zeros((NP, 128), jnp.float32)

    z, y, xin = _tc1(x, W_aff, b_aff, W0a, W_in, b_in)

    lg = _sc_logits(z, ei_r)

    pB = _sc_wagg32(y, ei_r, lg, zerosNP)
    hx, t = _gin0(pB, y, b0a, W0b, b0b, bn0_g, bn0_b, W1a, xin)

    pC = _sc_wagg128(t, ei_r, lg, zerosNP)
    hw = _gin1(pC, t, b1a, W1b, b1b, bn1_g, bn1_b, hx, Wc0a)

    p1 = _sc_unit_agg(hw, ei_r, zerosNP)
    gw = _rep_mid(p1, hw, bc0a, Wc0b, bc0b, bnc0_g, bnc0_b, Wc1a)

    p2 = _sc_unit_agg(gw, ei_r, zerosNP)
    g2 = _rep_final(p2, gw, bc1a, Wc1b, bc1b, bnc1_g, bnc1_b)
    return g2


# speedup vs baseline: 9.3067x; 1.3109x over previous
import functools

import jax
import jax.numpy as jnp
from jax import lax
from jax.experimental import pallas as pl
from jax.experimental.pallas import tpu as pltpu
from jax.experimental.pallas import tpu_sc as plsc

N = 10000
E = 320000
D = 128
NCOM = 4
CD = 32

NCOR = 2
NSUB = 16
LANE = 16

KW = 2
WINE = KW * 128
EP = 327680
NP = N + 112
RPS = NP // NSUB


def _sc_mesh():
    return plsc.VectorSubcoreMesh(core_axis_name="c", subcore_axis_name="s")


def _unit_agg_body(tab, ei_r, zeros, out, src_i, dst_i, rows_v, acc_sh, sem):
    c = lax.axis_index("c")
    s = lax.axis_index("s")
    pltpu.sync_copy(zeros.at[pl.ds(s * RPS, RPS)],
                    acc_sh.at[pl.ds(s * RPS, RPS)])
    plsc.subcore_barrier()

    wid = s * NCOR + c
    nw = EP // 32 // WINE

    def window(w, carry):
        row0 = wid * (nw * KW) + w * KW
        pltpu.sync_copy(ei_r.at[0, pl.ds(row0, KW)], src_i)
        pltpu.sync_copy(ei_r.at[1, pl.ds(row0, KW)], dst_i)
        for j in range(KW):
            pltpu.async_copy(tab.at[src_i.at[j]],
                             rows_v.at[pl.ds(j * 128, 128)], sem).wait()
        for j in range(KW):
            pltpu.sync_copy(rows_v.at[pl.ds(j * 128, 128)],
                            acc_sh.at[dst_i.at[j]], add=True)
        return carry

    lax.fori_loop(0, nw, window, 0, unroll=False)
    plsc.subcore_barrier()
    pltpu.sync_copy(acc_sh.at[pl.ds(s * RPS, RPS)],
                    out.at[c, pl.ds(s * RPS, RPS)])


@jax.jit
def _sc_unit_agg(tab, ei_r, zeros):
    return pl.kernel(
        _unit_agg_body,
        out_type=jax.ShapeDtypeStruct((2, NP, 128), jnp.float32),
        mesh=_sc_mesh(),
        scratch_types=[
            pltpu.VMEM((KW, 128), jnp.int32),
            pltpu.VMEM((KW, 128), jnp.int32),
            pltpu.VMEM((WINE, 128), jnp.float32),
            pltpu.VMEM_SHARED((NP, 128), jnp.float32),
            pltpu.SemaphoreType.DMA,
        ],
        compiler_params=pltpu.CompilerParams(use_tc_tiling_on_sc=False),
        name="sc_unit_agg",
    )(tab, ei_r, zeros)


def _make_wagg_body(tw, wine):
    kw = wine // 128

    def body(tab, ei_r, lg, zeros, out, src_i, dst_i, rows_v, msg_v, ew_v,
             acc_sh, sem):
        c = lax.axis_index("c")
        s = lax.axis_index("s")
        pltpu.sync_copy(zeros.at[pl.ds(s * RPS, RPS)],
                        acc_sh.at[pl.ds(s * RPS, RPS)])
        plsc.subcore_barrier()

        wid = s * NCOR + c
        nw = EP // 32 // wine

        def window(w, carry):
            e0 = wid * (nw * wine) + w * wine
            row0 = e0 // 128
            pltpu.sync_copy(ei_r.at[0, pl.ds(row0, kw)], src_i)
            pltpu.sync_copy(ei_r.at[1, pl.ds(row0, kw)], dst_i)
            for j in range(kw):
                pltpu.async_copy(tab.at[src_i.at[j]],
                                 rows_v.at[pl.ds(j * 128, 128)], sem).wait()
            pltpu.sync_copy(lg.at[:, pl.ds(e0, wine)],
                            ew_v.at[:, pl.ds(0, wine)])
            for g in range(wine // 16):
                sl = pl.ds(g * 16, 16)
                l0, l1, l2, l3 = (ew_v[0, sl], ew_v[1, sl], ew_v[2, sl],
                                  ew_v[3, sl])
                m = jnp.maximum(jnp.maximum(l0, l1), jnp.maximum(l2, l3))
                x0, x1 = jnp.exp(l0 - m), jnp.exp(l1 - m)
                x2, x3 = jnp.exp(l2 - m), jnp.exp(l3 - m)
                inv = 1.0 / (x0 + x1 + x2 + x3)
                ew_v[0, sl] = x0 * inv
                ew_v[1, sl] = x1 * inv
                ew_v[2, sl] = x2 * inv
                ew_v[3, sl] = x3 * inv

            def egroup(g, carry2):
                wv = [ew_v[k, pl.ds(g * 16, 16)] for k in range(NCOM)]
                for j in range(16):
                    e = g * 16 + j
                    if tw == 32:
                        ya = rows_v[e, pl.ds(0, 16)]
                        yb = rows_v[e, pl.ds(16, 16)]
                        for k in range(NCOM):
                            wk = wv[k][j]
                            msg_v[e, pl.ds(k * 32, 16)] = ya * wk
                            msg_v[e, pl.ds(k * 32 + 16, 16)] = yb * wk
                    else:
                        for k in range(NCOM):
                            wk = wv[k][j]
                            a = pl.ds(k * 32, 16)
                            b = pl.ds(k * 32 + 16, 16)
                            msg_v[e, a] = rows_v[e, a] * wk
                            msg_v[e, b] = rows_v[e, b] * wk
                return carry2

            lax.fori_loop(0, wine // 16, egroup, 0, unroll=False)
            for j in range(kw):
                pltpu.sync_copy(msg_v.at[pl.ds(j * 128, 128)],
                                acc_sh.at[dst_i.at[j]], add=True)
            return carry

        lax.fori_loop(0, nw, window, 0, unroll=False)
        plsc.subcore_barrier()
        pltpu.sync_copy(acc_sh.at[pl.ds(s * RPS, RPS)],
                        out.at[c, pl.ds(s * RPS, RPS)])

    return body


def _make_wagg(tw, wine):
    kw = wine // 128
    rows_shape = (wine, tw) if tw == 32 else (wine, 128)

    @jax.jit
    def run(tab, ei_r, lg, zeros):
        return pl.kernel(
            _make_wagg_body(tw, wine),
            out_type=jax.ShapeDtypeStruct((2, NP, 128), jnp.float32),
            mesh=_sc_mesh(),
            scratch_types=[
                pltpu.VMEM((kw, 128), jnp.int32),
                pltpu.VMEM((kw, 128), jnp.int32),
                pltpu.VMEM(rows_shape, jnp.float32),
                pltpu.VMEM((wine, 128), jnp.float32),
                pltpu.VMEM((4, wine + 16), jnp.float32),
                pltpu.VMEM_SHARED((NP, 128), jnp.float32),
                pltpu.SemaphoreType.DMA,
            ],
            compiler_params=pltpu.CompilerParams(use_tc_tiling_on_sc=False),
            name=f"sc_wagg{tw}",
        )(tab, ei_r, lg, zeros)

    return run


_sc_wagg32 = _make_wagg(32, 128)
_sc_wagg128 = _make_wagg(128, 128)


WINL = 256
KL = WINL // 128


def _logits_body(z, ei_r, out, src_i, dst_i, zs_v, zd_v, lgo_v, sem):
    c = lax.axis_index("c")
    s = lax.axis_index("s")
    wid = s * NCOR + c
    nw = EP // 32 // WINL
    lanes = jnp.arange(16, dtype=jnp.int32)

    def window(w, carry):
        e0 = wid * (nw * WINL) + w * WINL
        row0 = e0 // 128
        pltpu.sync_copy(ei_r.at[0, pl.ds(row0, KL)], src_i)
        pltpu.sync_copy(ei_r.at[1, pl.ds(row0, KL)], dst_i)
        descs = []
        for j in range(KL):
            descs.append(pltpu.async_copy(
                z.at[src_i.at[j]], zs_v.at[pl.ds(j * 128, 128)], sem))
            descs.append(pltpu.async_copy(
                z.at[dst_i.at[j]], zd_v.at[pl.ds(j * 128, 128)], sem))
        for dsc in descs:
            dsc.wait()

        def gblock(g, carry2):
            acc = [jnp.zeros((16,), jnp.float32) for _ in range(NCOM)]
            for j in range(16):
                e = g * 16 + j
                for k in range(NCOM):
                    a = zs_v[e, pl.ds(k * 32, 16)] * zd_v[e, pl.ds(k * 32, 16)]
                    b = (zs_v[e, pl.ds(k * 32 + 16, 16)]
                         * zd_v[e, pl.ds(k * 32 + 16, 16)])
                    sk = jnp.sum(a + b)
                    acc[k] = jnp.where(lanes == j, sk, acc[k])
            for k in range(NCOM):
                lgo_v[k, pl.ds(g * 16, 16)] = acc[k]
            return carry2

        lax.fori_loop(0, WINL // 16, gblock, 0, unroll=False)
        pltpu.sync_copy(lgo_v, out.at[:, pl.ds(e0, WINL)])
        return carry

    lax.fori_loop(0, nw, window, 0, unroll=False)


@jax.jit
def _sc_logits(z, ei_r):
    return pl.kernel(
        _logits_body,
        out_type=jax.ShapeDtypeStruct((4, EP), jnp.float32),
        mesh=_sc_mesh(),
        scratch_types=[
            pltpu.VMEM((KL, 128), jnp.int32),
            pltpu.VMEM((KL, 128), jnp.int32),
            pltpu.VMEM((WINL, 128), jnp.float32),
            pltpu.VMEM((WINL, 128), jnp.float32),
            pltpu.VMEM((4, WINL), jnp.float32),
            pltpu.SemaphoreType.DMA,
        ],
        compiler_params=pltpu.CompilerParams(use_tc_tiling_on_sc=False,
                                             needs_layout_passes=False),
        name="sc_logits",
    )(z, ei_r)


def _bn(h, g, b):
    return h * (g / jnp.sqrt(1.0 + 1e-5)) + b


_BNS = float(1.0 / (1.0 + 1e-5) ** 0.5)


def _tc1_body(x_r, W_aff_r, b_aff_r, W0a_r, W_in_r, b_in_r, z_o, y_o, xin_o):
    x = x_r[...]
    z_o[...] = jnp.maximum(
        jnp.dot(x, W_aff_r[...], preferred_element_type=jnp.float32)
        + b_aff_r[...], 0.0)
    y_o[...] = jnp.dot(x, W0a_r[...], preferred_element_type=jnp.float32)
    xin_o[...] = jnp.dot(x, W_in_r[...],
                         preferred_element_type=jnp.float32) + b_in_r[...]


@jax.jit
def _tc1(x, W_aff, b_aff, W0a, W_in, b_in):
    return pl.pallas_call(
        _tc1_body,
        out_shape=[jax.ShapeDtypeStruct((N, D), jnp.float32),
                   jax.ShapeDtypeStruct((N, CD), jnp.float32),
                   jax.ShapeDtypeStruct((N, D), jnp.float32)],
    )(x, W_aff, b_aff.reshape(1, D), W0a, W_in, b_in.reshape(1, D))


def _gin0_body(p_r, y4_r, ba_r, Wb_r, bb_r, bng_r, bnb_r, Wn_r, xin_r,
               hx_o, t_o):
    agg = p_r[0, :N] + p_r[1, :N]
    u = jnp.maximum(y4_r[...] + agg + ba_r[...], 0.0)
    hpre = jnp.dot(u, Wb_r[...], preferred_element_type=jnp.float32) \
        + bb_r[...]
    h = jnp.maximum(hpre * (bng_r[...] * _BNS) + bnb_r[...], 0.0)
    hx_o[...] = h + xin_r[...]
    t_o[...] = jnp.dot(h, Wn_r[...], preferred_element_type=jnp.float32)


@jax.jit
def _gin0(p, y, b0a, W0b, b0b, bn0_g, bn0_b, W1a, xin):
    eye4 = jnp.eye(NCOM, dtype=jnp.float32)
    y4 = jnp.tile(y, (1, NCOM))
    return pl.pallas_call(
        _gin0_body,
        out_shape=[jax.ShapeDtypeStruct((N, D), jnp.float32),
                   jax.ShapeDtypeStruct((N, D), jnp.float32)],
    )(p, y4, jnp.tile(b0a, NCOM).reshape(1, D), jnp.kron(eye4, W0b),
      jnp.tile(b0b, NCOM).reshape(1, D), jnp.tile(bn0_g, NCOM).reshape(1, D),
      jnp.tile(bn0_b, NCOM).reshape(1, D), jnp.kron(eye4, W1a), xin)


def _gin1_body(p_r, t_r, ba_r, Wb_r, bb_r, bng_r, bnb_r, hx_r, Wn_r, hw_o):
    agg = p_r[0, :N] + p_r[1, :N]
    u = jnp.maximum(t_r[...] + agg + ba_r[...], 0.0)
    hpre = jnp.dot(u, Wb_r[...], preferred_element_type=jnp.float32) \
        + bb_r[...]
    h2 = jnp.maximum(hpre * (bng_r[...] * _BNS) + bnb_r[...], 0.0)
    h = hx_r[...] + h2
    hw_o[...] = jnp.dot(h, Wn_r[...], preferred_element_type=jnp.float32)


@jax.jit
def _gin1(p, t, b1a, W1b, b1b, bn1_g, bn1_b, hx, Wc0a):
    eye4 = jnp.eye(NCOM, dtype=jnp.float32)
    return pl.pallas_call(
        _gin1_body,
        out_shape=jax.ShapeDtypeStruct((N, D), jnp.float32),
    )(p, t, jnp.tile(b1a, NCOM).reshape(1, D), jnp.kron(eye4, W1b),
      jnp.tile(b1b, NCOM).reshape(1, D), jnp.tile(bn1_g, NCOM).reshape(1, D),
      jnp.tile(bn1_b, NCOM).reshape(1, D), hx, Wc0a)


def _rep_mid_body(p_r, base_r, ba_r, Wb_r, bb_r, bng_r, bnb_r, Wn_r, gw_o):
    agg = p_r[0, :N] + p_r[1, :N]
    u = jnp.maximum(base_r[...] + agg + ba_r[...], 0.0)
    hpre = jnp.dot(u, Wb_r[...], preferred_element_type=jnp.float32) \
        + bb_r[...]
    g1 = jnp.maximum(hpre * (bng_r[...] * _BNS) + bnb_r[...], 0.0)
    gw_o[...] = jnp.dot(g1, Wn_r[...], preferred_element_type=jnp.float32)


@jax.jit
def _rep_mid(p, base, ba, Wb, bb, bng, bnb, Wn):
    return pl.pallas_call(
        _rep_mid_body,
        out_shape=jax.ShapeDtypeStruct((N, D), jnp.float32),
    )(p, base, ba.reshape(1, D), Wb, bb.reshape(1, D), bng.reshape(1, D),
      bnb.reshape(1, D), Wn)


def _rep_final_body(p_r, base_r, ba_r, Wb_r, bb_r, bng_r, bnb_r, g2_o):
    agg = p_r[0, :N] + p_r[1, :N]
    u = jnp.maximum(base_r[...] + agg + ba_r[...], 0.0)
    hpre = jnp.dot(u, Wb_r[...], preferred_element_type=jnp.float32) \
        + bb_r[...]
    g2_o[...] = jnp.maximum(hpre * (bng_r[...] * _BNS) + bnb_r[...], 0.0)


@jax.jit
def _rep_final(p, base, ba, Wb, bb, bng, bnb):
    return pl.pallas_call(
        _rep_final_body,
        out_shape=jax.ShapeDtypeStruct((N, D), jnp.float32),
    )(p, base, ba.reshape(1, D), Wb, bb.reshape(1, D), bng.reshape(1, D),
      bnb.reshape(1, D))


def kernel(x, edge_index, W_aff, b_aff, W0a, b0a, W0b, b0b, bn0_g, bn0_b,
           W1a, b1a, W1b, b1b, bn1_g, bn1_b, W_in, b_in,
           Wc0a, bc0a, Wc0b, bc0b, bnc0_g, bnc0_b,
           Wc1a, bc1a, Wc1b, bc1b, bnc1_g, bnc1_b):
    src, dst = edge_index[0], edge_index[1]

    npad = EP - E
    pad_lane = (jnp.arange(npad, dtype=jnp.int32) % 112)
    ei_r = jnp.concatenate([
        jnp.stack([src, dst]),
        jnp.stack([pad_lane, N + pad_lane]),
    ], axis=1).reshape(2, EP // 128, 128)
    zerosNP = jnp.zeros((NP, 128), jnp.float32)

    z, y, xin = _tc1(x, W_aff, b_aff, W0a, W_in, b_in)

    lg = _sc_logits(z, ei_r)

    pB = _sc_wagg32(y, ei_r, lg, zerosNP)
    hx, t = _gin0(pB, y, b0a, W0b, b0b, bn0_g, bn0_b, W1a, xin)

    pC = _sc_wagg128(t, ei_r, lg, zerosNP)
    hw = _gin1(pC, t, b1a, W1b, b1b, bn1_g, bn1_b, hx, Wc0a)

    p1 = _sc_unit_agg(hw, ei_r, zerosNP)
    gw = _rep_mid(p1, hw, bc0a, Wc0b, bc0b, bnc0_g, bnc0_b, Wc1a)

    p2 = _sc_unit_agg(gw, ei_r, zerosNP)
    g2 = _rep_final(p2, gw, bc1a, Wc1b, bc1b, bnc1_g, bnc1_b)
    return g2


# speedup vs baseline: 10.1273x; 1.0882x over previous
import functools

import jax
import jax.numpy as jnp
from jax import lax
from jax.experimental import pallas as pl
from jax.experimental.pallas import tpu as pltpu
from jax.experimental.pallas import tpu_sc as plsc

N = 10000
E = 320000
D = 128
NCOM = 4
CD = 32

NCOR = 2
NSUB = 16
LANE = 16

KW = 2
WINE = KW * 128
EP = 327680
NP = N + 112
RPS = NP // NSUB


def _sc_mesh():
    return plsc.VectorSubcoreMesh(core_axis_name="c", subcore_axis_name="s")


WU = 128


def _unit_agg_body(tab, ei_r, zeros, out, src_i, dst_i, rows_v, acc_sh,
                   sem0, sem1):
    c = lax.axis_index("c")
    s = lax.axis_index("s")
    pltpu.sync_copy(zeros.at[pl.ds(s * RPS, RPS)],
                    acc_sh.at[pl.ds(s * RPS, RPS)])
    plsc.subcore_barrier()

    wid = s * NCOR + c
    nw = EP // 32 // WU
    row_base = wid * nw
    sems = (sem0, sem1)

    def stage_and_fire(w, b):
        pltpu.sync_copy(ei_r.at[0, pl.ds(row_base + w, 1)], src_i.at[b])
        pltpu.sync_copy(ei_r.at[1, pl.ds(row_base + w, 1)], dst_i.at[b])
        return pltpu.async_copy(tab.at[src_i.at[b, 0]], rows_v.at[b], sems[b])

    def drain_and_scatter(b):
        pltpu.make_async_copy(tab.at[src_i.at[b, 0]], rows_v.at[b],
                              sems[b]).wait()
        pltpu.sync_copy(rows_v.at[b], acc_sh.at[dst_i.at[b, 0]], add=True)

    stage_and_fire(0, 0)

    def pair(w2, carry):
        w0 = 2 * w2
        stage_and_fire(w0 + 1, 1)
        drain_and_scatter(0)

        @pl.when(w0 + 2 < nw)
        def _():
            stage_and_fire(w0 + 2, 0)

        drain_and_scatter(1)
        return carry

    lax.fori_loop(0, nw // 2, pair, 0, unroll=False)
    plsc.subcore_barrier()
    pltpu.sync_copy(acc_sh.at[pl.ds(s * RPS, RPS)],
                    out.at[c, pl.ds(s * RPS, RPS)])


@jax.jit
def _sc_unit_agg(tab, ei_r, zeros):
    return pl.kernel(
        _unit_agg_body,
        out_type=jax.ShapeDtypeStruct((2, NP, 128), jnp.float32),
        mesh=_sc_mesh(),
        scratch_types=[
            pltpu.VMEM((2, 1, 128), jnp.int32),
            pltpu.VMEM((2, 1, 128), jnp.int32),
            pltpu.VMEM((2, WU, 128), jnp.float32),
            pltpu.VMEM_SHARED((NP, 128), jnp.float32),
            pltpu.SemaphoreType.DMA,
            pltpu.SemaphoreType.DMA,
        ],
        compiler_params=pltpu.CompilerParams(use_tc_tiling_on_sc=False),
        name="sc_unit_agg",
    )(tab, ei_r, zeros)


def _make_wagg_body(tw, wine):
    kw = wine // 128

    def body(tab, ei_r, lg, zeros, out, src_i, dst_i, rows_v, msg_v, ew_v,
             acc_sh, sem):
        c = lax.axis_index("c")
        s = lax.axis_index("s")
        pltpu.sync_copy(zeros.at[pl.ds(s * RPS, RPS)],
                        acc_sh.at[pl.ds(s * RPS, RPS)])
        plsc.subcore_barrier()

        wid = s * NCOR + c
        nw = EP // 32 // wine

        def window(w, carry):
            e0 = wid * (nw * wine) + w * wine
            row0 = e0 // 128
            pltpu.sync_copy(ei_r.at[0, pl.ds(row0, kw)], src_i)
            pltpu.sync_copy(ei_r.at[1, pl.ds(row0, kw)], dst_i)
            for j in range(kw):
                pltpu.async_copy(tab.at[src_i.at[j]],
                                 rows_v.at[pl.ds(j * 128, 128)], sem).wait()
            pltpu.sync_copy(lg.at[:, pl.ds(e0, wine)],
                            ew_v.at[:, pl.ds(0, wine)])
            for g in range(wine // 16):
                sl = pl.ds(g * 16, 16)
                l0, l1, l2, l3 = (ew_v[0, sl], ew_v[1, sl], ew_v[2, sl],
                                  ew_v[3, sl])
                m = jnp.maximum(jnp.maximum(l0, l1), jnp.maximum(l2, l3))
                x0, x1 = jnp.exp(l0 - m), jnp.exp(l1 - m)
                x2, x3 = jnp.exp(l2 - m), jnp.exp(l3 - m)
                inv = 1.0 / (x0 + x1 + x2 + x3)
                ew_v[0, sl] = x0 * inv
                ew_v[1, sl] = x1 * inv
                ew_v[2, sl] = x2 * inv
                ew_v[3, sl] = x3 * inv

            def egroup(g, carry2):
                wv = [ew_v[k, pl.ds(g * 16, 16)] for k in range(NCOM)]
                for j in range(16):
                    e = g * 16 + j
                    if tw == 32:
                        ya = rows_v[e, pl.ds(0, 16)]
                        yb = rows_v[e, pl.ds(16, 16)]
                        for k in range(NCOM):
                            wk = wv[k][j]
                            msg_v[e, pl.ds(k * 32, 16)] = ya * wk
                            msg_v[e, pl.ds(k * 32 + 16, 16)] = yb * wk
                    else:
                        for k in range(NCOM):
                            wk = wv[k][j]
                            a = pl.ds(k * 32, 16)
                            b = pl.ds(k * 32 + 16, 16)
                            msg_v[e, a] = rows_v[e, a] * wk
                            msg_v[e, b] = rows_v[e, b] * wk
                return carry2

            lax.fori_loop(0, wine // 16, egroup, 0, unroll=False)
            for j in range(kw):
                pltpu.sync_copy(msg_v.at[pl.ds(j * 128, 128)],
                                acc_sh.at[dst_i.at[j]], add=True)
            return carry

        lax.fori_loop(0, nw, window, 0, unroll=False)
        plsc.subcore_barrier()
        pltpu.sync_copy(acc_sh.at[pl.ds(s * RPS, RPS)],
                        out.at[c, pl.ds(s * RPS, RPS)])

    return body


def _make_wagg(tw, wine):
    kw = wine // 128
    rows_shape = (wine, tw) if tw == 32 else (wine, 128)

    @jax.jit
    def run(tab, ei_r, lg, zeros):
        return pl.kernel(
            _make_wagg_body(tw, wine),
            out_type=jax.ShapeDtypeStruct((2, NP, 128), jnp.float32),
            mesh=_sc_mesh(),
            scratch_types=[
                pltpu.VMEM((kw, 128), jnp.int32),
                pltpu.VMEM((kw, 128), jnp.int32),
                pltpu.VMEM(rows_shape, jnp.float32),
                pltpu.VMEM((wine, 128), jnp.float32),
                pltpu.VMEM((4, wine + 16), jnp.float32),
                pltpu.VMEM_SHARED((NP, 128), jnp.float32),
                pltpu.SemaphoreType.DMA,
            ],
            compiler_params=pltpu.CompilerParams(use_tc_tiling_on_sc=False),
            name=f"sc_wagg{tw}",
        )(tab, ei_r, lg, zeros)

    return run


_sc_wagg32 = _make_wagg(32, 128)
_sc_wagg128 = _make_wagg(128, 128)


WINL = 256
KL = WINL // 128


def _logits_body(z, ei_r, out, src_i, dst_i, zs_v, zd_v, lgo_v, sem):
    c = lax.axis_index("c")
    s = lax.axis_index("s")
    wid = s * NCOR + c
    nw = EP // 32 // WINL
    lanes = jnp.arange(16, dtype=jnp.int32)

    def window(w, carry):
        e0 = wid * (nw * WINL) + w * WINL
        row0 = e0 // 128
        pltpu.sync_copy(ei_r.at[0, pl.ds(row0, KL)], src_i)
        pltpu.sync_copy(ei_r.at[1, pl.ds(row0, KL)], dst_i)
        descs = []
        for j in range(KL):
            descs.append(pltpu.async_copy(
                z.at[src_i.at[j]], zs_v.at[pl.ds(j * 128, 128)], sem))
            descs.append(pltpu.async_copy(
                z.at[dst_i.at[j]], zd_v.at[pl.ds(j * 128, 128)], sem))
        for dsc in descs:
            dsc.wait()

        def gblock(g, carry2):
            acc = [jnp.zeros((16,), jnp.float32) for _ in range(NCOM)]
            for j in range(16):
                e = g * 16 + j
                for k in range(NCOM):
                    a = zs_v[e, pl.ds(k * 32, 16)] * zd_v[e, pl.ds(k * 32, 16)]
                    b = (zs_v[e, pl.ds(k * 32 + 16, 16)]
                         * zd_v[e, pl.ds(k * 32 + 16, 16)])
                    sk = jnp.sum(a + b)
                    acc[k] = jnp.where(lanes == j, sk, acc[k])
            for k in range(NCOM):
                lgo_v[k, pl.ds(g * 16, 16)] = acc[k]
            return carry2

        lax.fori_loop(0, WINL // 16, gblock, 0, unroll=False)
        pltpu.sync_copy(lgo_v, out.at[:, pl.ds(e0, WINL)])
        return carry

    lax.fori_loop(0, nw, window, 0, unroll=False)


@jax.jit
def _sc_logits(z, ei_r):
    return pl.kernel(
        _logits_body,
        out_type=jax.ShapeDtypeStruct((4, EP), jnp.float32),
        mesh=_sc_mesh(),
        scratch_types=[
            pltpu.VMEM((KL, 128), jnp.int32),
            pltpu.VMEM((KL, 128), jnp.int32),
            pltpu.VMEM((WINL, 128), jnp.float32),
            pltpu.VMEM((WINL, 128), jnp.float32),
            pltpu.VMEM((4, WINL), jnp.float32),
            pltpu.SemaphoreType.DMA,
        ],
        compiler_params=pltpu.CompilerParams(use_tc_tiling_on_sc=False,
                                             needs_layout_passes=False),
        name="sc_logits",
    )(z, ei_r)


def _bn(h, g, b):
    return h * (g / jnp.sqrt(1.0 + 1e-5)) + b


_BNS = float(1.0 / (1.0 + 1e-5) ** 0.5)


def _tc1_body(x_r, W_aff_r, b_aff_r, W0a_r, W_in_r, b_in_r, z_o, y_o, xin_o):
    x = x_r[...]
    z_o[...] = jnp.maximum(
        jnp.dot(x, W_aff_r[...], preferred_element_type=jnp.float32)
        + b_aff_r[...], 0.0)
    y_o[...] = jnp.dot(x, W0a_r[...], preferred_element_type=jnp.float32)
    xin_o[...] = jnp.dot(x, W_in_r[...],
                         preferred_element_type=jnp.float32) + b_in_r[...]


@jax.jit
def _tc1(x, W_aff, b_aff, W0a, W_in, b_in):
    return pl.pallas_call(
        _tc1_body,
        out_shape=[jax.ShapeDtypeStruct((N, D), jnp.float32),
                   jax.ShapeDtypeStruct((N, CD), jnp.float32),
                   jax.ShapeDtypeStruct((N, D), jnp.float32)],
    )(x, W_aff, b_aff.reshape(1, D), W0a, W_in, b_in.reshape(1, D))


def _gin0_body(p_r, y4_r, ba_r, Wb_r, bb_r, bng_r, bnb_r, Wn_r, xin_r,
               hx_o, t_o):
    agg = p_r[0, :N] + p_r[1, :N]
    u = jnp.maximum(y4_r[...] + agg + ba_r[...], 0.0)
    hpre = jnp.dot(u, Wb_r[...], preferred_element_type=jnp.float32) \
        + bb_r[...]
    h = jnp.maximum(hpre * (bng_r[...] * _BNS) + bnb_r[...], 0.0)
    hx_o[...] = h + xin_r[...]
    t_o[...] = jnp.dot(h, Wn_r[...], preferred_element_type=jnp.float32)


@jax.jit
def _gin0(p, y, b0a, W0b, b0b, bn0_g, bn0_b, W1a, xin):
    eye4 = jnp.eye(NCOM, dtype=jnp.float32)
    y4 = jnp.tile(y, (1, NCOM))
    return pl.pallas_call(
        _gin0_body,
        out_shape=[jax.ShapeDtypeStruct((N, D), jnp.float32),
                   jax.ShapeDtypeStruct((N, D), jnp.float32)],
    )(p, y4, jnp.tile(b0a, NCOM).reshape(1, D), jnp.kron(eye4, W0b),
      jnp.tile(b0b, NCOM).reshape(1, D), jnp.tile(bn0_g, NCOM).reshape(1, D),
      jnp.tile(bn0_b, NCOM).reshape(1, D), jnp.kron(eye4, W1a), xin)


def _gin1_body(p_r, t_r, ba_r, Wb_r, bb_r, bng_r, bnb_r, hx_r, Wn_r, hw_o):
    agg = p_r[0, :N] + p_r[1, :N]
    u = jnp.maximum(t_r[...] + agg + ba_r[...], 0.0)
    hpre = jnp.dot(u, Wb_r[...], preferred_element_type=jnp.float32) \
        + bb_r[...]
    h2 = jnp.maximum(hpre * (bng_r[...] * _BNS) + bnb_r[...], 0.0)
    h = hx_r[...] + h2
    hw_o[...] = jnp.dot(h, Wn_r[...], preferred_element_type=jnp.float32)


@jax.jit
def _gin1(p, t, b1a, W1b, b1b, bn1_g, bn1_b, hx, Wc0a):
    eye4 = jnp.eye(NCOM, dtype=jnp.float32)
    return pl.pallas_call(
        _gin1_body,
        out_shape=jax.ShapeDtypeStruct((N, D), jnp.float32),
    )(p, t, jnp.tile(b1a, NCOM).reshape(1, D), jnp.kron(eye4, W1b),
      jnp.tile(b1b, NCOM).reshape(1, D), jnp.tile(bn1_g, NCOM).reshape(1, D),
      jnp.tile(bn1_b, NCOM).reshape(1, D), hx, Wc0a)


def _rep_mid_body(p_r, base_r, ba_r, Wb_r, bb_r, bng_r, bnb_r, Wn_r, gw_o):
    agg = p_r[0, :N] + p_r[1, :N]
    u = jnp.maximum(base_r[...] + agg + ba_r[...], 0.0)
    hpre = jnp.dot(u, Wb_r[...], preferred_element_type=jnp.float32) \
        + bb_r[...]
    g1 = jnp.maximum(hpre * (bng_r[...] * _BNS) + bnb_r[...], 0.0)
    gw_o[...] = jnp.dot(g1, Wn_r[...], preferred_element_type=jnp.float32)


@jax.jit
def _rep_mid(p, base, ba, Wb, bb, bng, bnb, Wn):
    return pl.pallas_call(
        _rep_mid_body,
        out_shape=jax.ShapeDtypeStruct((N, D), jnp.float32),
    )(p, base, ba.reshape(1, D), Wb, bb.reshape(1, D), bng.reshape(1, D),
      bnb.reshape(1, D), Wn)


def _rep_final_body(p_r, base_r, ba_r, Wb_r, bb_r, bng_r, bnb_r, g2_o):
    agg = p_r[0, :N] + p_r[1, :N]
    u = jnp.maximum(base_r[...] + agg + ba_r[...], 0.0)
    hpre = jnp.dot(u, Wb_r[...], preferred_element_type=jnp.float32) \
        + bb_r[...]
    g2_o[...] = jnp.maximum(hpre * (bng_r[...] * _BNS) + bnb_r[...], 0.0)


@jax.jit
def _rep_final(p, base, ba, Wb, bb, bng, bnb):
    return pl.pallas_call(
        _rep_final_body,
        out_shape=jax.ShapeDtypeStruct((N, D), jnp.float32),
    )(p, base, ba.reshape(1, D), Wb, bb.reshape(1, D), bng.reshape(1, D),
      bnb.reshape(1, D))


def kernel(x, edge_index, W_aff, b_aff, W0a, b0a, W0b, b0b, bn0_g, bn0_b,
           W1a, b1a, W1b, b1b, bn1_g, bn1_b, W_in, b_in,
           Wc0a, bc0a, Wc0b, bc0b, bnc0_g, bnc0_b,
           Wc1a, bc1a, Wc1b, bc1b, bnc1_g, bnc1_b):
    src, dst = edge_index[0], edge_index[1]

    npad = EP - E
    pad_lane = (jnp.arange(npad, dtype=jnp.int32) % 112)
    ei_r = jnp.concatenate([
        jnp.stack([src, dst]),
        jnp.stack([pad_lane, N + pad_lane]),
    ], axis=1).reshape(2, EP // 128, 128)
    zerosNP = jnp.zeros((NP, 128), jnp.float32)

    z, y, xin = _tc1(x, W_aff, b_aff, W0a, W_in, b_in)

    lg = _sc_logits(z, ei_r)

    pB = _sc_wagg32(y, ei_r, lg, zerosNP)
    hx, t = _gin0(pB, y, b0a, W0b, b0b, bn0_g, bn0_b, W1a, xin)

    pC = _sc_wagg128(t, ei_r, lg, zerosNP)
    hw = _gin1(pC, t, b1a, W1b, b1b, bn1_g, bn1_b, hx, Wc0a)

    p1 = _sc_unit_agg(hw, ei_r, zerosNP)
    gw = _rep_mid(p1, hw, bc0a, Wc0b, bc0b, bnc0_g, bnc0_b, Wc1a)

    p2 = _sc_unit_agg(gw, ei_r, zerosNP)
    g2 = _rep_final(p2, gw, bc1a, Wc1b, bc1b, bnc1_g, bnc1_b)
    return g2


# speedup vs baseline: 12.3077x; 1.2153x over previous
import functools

import jax
import jax.numpy as jnp
from jax import lax
from jax.experimental import pallas as pl
from jax.experimental.pallas import tpu as pltpu
from jax.experimental.pallas import tpu_sc as plsc

N = 10000
E = 320000
D = 128
NCOM = 4
CD = 32

NCOR = 2
NSUB = 16
LANE = 16

KW = 2
WINE = KW * 128
EP = 327680
NP = N + 112
RPS = NP // NSUB


def _sc_mesh():
    return plsc.VectorSubcoreMesh(core_axis_name="c", subcore_axis_name="s")


WU = 128


def _unit_agg_body(tab, ei_r, zeros, out, src_i, dst_i, rows_v, acc_sh,
                   sem0, sem1):
    c = lax.axis_index("c")
    s = lax.axis_index("s")
    pltpu.sync_copy(zeros.at[pl.ds(s * RPS, RPS)],
                    acc_sh.at[pl.ds(s * RPS, RPS)])
    plsc.subcore_barrier()

    wid = s * NCOR + c
    nw = EP // 32 // WU
    row_base = wid * nw
    sems = (sem0, sem1)

    def stage_and_fire(w, b):
        pltpu.sync_copy(ei_r.at[0, pl.ds(row_base + w, 1)], src_i.at[b])
        pltpu.sync_copy(ei_r.at[1, pl.ds(row_base + w, 1)], dst_i.at[b])
        return pltpu.async_copy(tab.at[src_i.at[b, 0]], rows_v.at[b], sems[b])

    def drain_and_scatter(b):
        pltpu.make_async_copy(tab.at[src_i.at[b, 0]], rows_v.at[b],
                              sems[b]).wait()
        pltpu.sync_copy(rows_v.at[b], acc_sh.at[dst_i.at[b, 0]], add=True)

    stage_and_fire(0, 0)

    def pair(w2, carry):
        w0 = 2 * w2
        stage_and_fire(w0 + 1, 1)
        drain_and_scatter(0)

        @pl.when(w0 + 2 < nw)
        def _():
            stage_and_fire(w0 + 2, 0)

        drain_and_scatter(1)
        return carry

    lax.fori_loop(0, nw // 2, pair, 0, unroll=False)
    plsc.subcore_barrier()
    pltpu.sync_copy(acc_sh.at[pl.ds(s * RPS, RPS)],
                    out.at[c, pl.ds(s * RPS, RPS)])


@jax.jit
def _sc_unit_agg(tab, ei_r, zeros):
    return pl.kernel(
        _unit_agg_body,
        out_type=jax.ShapeDtypeStruct((2, NP, 128), jnp.float32),
        mesh=_sc_mesh(),
        scratch_types=[
            pltpu.VMEM((2, 1, 128), jnp.int32),
            pltpu.VMEM((2, 1, 128), jnp.int32),
            pltpu.VMEM((2, WU, 128), jnp.float32),
            pltpu.VMEM_SHARED((NP, 128), jnp.float32),
            pltpu.SemaphoreType.DMA,
            pltpu.SemaphoreType.DMA,
        ],
        compiler_params=pltpu.CompilerParams(use_tc_tiling_on_sc=False),
        name="sc_unit_agg",
    )(tab, ei_r, zeros)


WW = 128


def _make_wagg_body(tw):
    def body(tab, ei_r, lg, zeros, out, src_i, dst_i, rows_v, msg_v, ew_v,
             acc_sh, semg0, semg1, sems0, sems1):
        c = lax.axis_index("c")
        s = lax.axis_index("s")
        pltpu.sync_copy(zeros.at[pl.ds(s * RPS, RPS)],
                        acc_sh.at[pl.ds(s * RPS, RPS)])
        plsc.subcore_barrier()

        wid = s * NCOR + c
        nw = EP // 32 // WW
        row_base = wid * nw
        semg = (semg0, semg1)
        sems = (sems0, sems1)
        gtgt = msg_v if tw == 128 else rows_v

        def stage_and_fire(w, b):
            pltpu.sync_copy(ei_r.at[0, pl.ds(row_base + w, 1)], src_i.at[b])
            pltpu.sync_copy(ei_r.at[1, pl.ds(row_base + w, 1)],
                            dst_i.at[w & 3])
            pltpu.sync_copy(lg.at[:, pl.ds((row_base + w) * 128, WW)],
                            ew_v.at[b, :, pl.ds(0, WW)])
            return pltpu.async_copy(tab.at[src_i.at[b, 0]], gtgt.at[b],
                                    semg[b])

        def drain_gather(b):
            pltpu.make_async_copy(tab.at[src_i.at[b, 0]], gtgt.at[b],
                                  semg[b]).wait()

        def drain_scatter(b):
            pltpu.make_async_copy(msg_v.at[b], acc_sh.at[dst_i.at[b, 0]],
                                  sems[b]).wait()

        def compute_and_fire(w, b):
            for g in range(WW // 16):
                sl = pl.ds(g * 16, 16)
                l0, l1, l2, l3 = (ew_v[b, 0, sl], ew_v[b, 1, sl],
                                  ew_v[b, 2, sl], ew_v[b, 3, sl])
                m = jnp.maximum(jnp.maximum(l0, l1), jnp.maximum(l2, l3))
                x0, x1 = jnp.exp(l0 - m), jnp.exp(l1 - m)
                x2, x3 = jnp.exp(l2 - m), jnp.exp(l3 - m)
                inv = 1.0 / (x0 + x1 + x2 + x3)
                ew_v[b, 0, sl] = x0 * inv
                ew_v[b, 1, sl] = x1 * inv
                ew_v[b, 2, sl] = x2 * inv
                ew_v[b, 3, sl] = x3 * inv

            def egroup(g, carry2):
                wv = [ew_v[b, k, pl.ds(g * 16, 16)] for k in range(NCOM)]
                for j in range(16):
                    e = g * 16 + j
                    if tw == 32:
                        ya = rows_v[b, e, pl.ds(0, 16)]
                        yb = rows_v[b, e, pl.ds(16, 16)]
                        for k in range(NCOM):
                            wk = wv[k][j]
                            msg_v[b, e, pl.ds(k * 32, 16)] = ya * wk
                            msg_v[b, e, pl.ds(k * 32 + 16, 16)] = yb * wk
                    else:
                        for k in range(NCOM):
                            wk = wv[k][j]
                            a = pl.ds(k * 32, 16)
                            bb = pl.ds(k * 32 + 16, 16)
                            msg_v[b, e, a] = msg_v[b, e, a] * wk
                            msg_v[b, e, bb] = msg_v[b, e, bb] * wk
                return carry2

            lax.fori_loop(0, WW // 16, egroup, 0, unroll=False)
            pltpu.async_copy(msg_v.at[b], acc_sh.at[dst_i.at[w & 3, 0]],
                             sems[b], add=True)

        stage_and_fire(0, 0)

        def pair(w2, carry):
            w0 = 2 * w2
            stage_and_fire(w0 + 1, 1)
            drain_gather(0)

            @pl.when(w2 >= 1)
            def _():
                drain_scatter(0)

            compute_and_fire(w0, 0)

            @pl.when(w0 + 2 < nw)
            def _():
                stage_and_fire(w0 + 2, 0)

            drain_gather(1)

            @pl.when(w2 >= 1)
            def _():
                drain_scatter(1)

            compute_and_fire(w0 + 1, 1)
            return carry

        lax.fori_loop(0, nw // 2, pair, 0, unroll=False)
        drain_scatter(0)
        drain_scatter(1)
        plsc.subcore_barrier()
        pltpu.sync_copy(acc_sh.at[pl.ds(s * RPS, RPS)],
                        out.at[c, pl.ds(s * RPS, RPS)])

    return body


def _make_wagg(tw):
    rows_shape = (2, WW, tw) if tw == 32 else (2, 1, 128)

    @jax.jit
    def run(tab, ei_r, lg, zeros):
        return pl.kernel(
            _make_wagg_body(tw),
            out_type=jax.ShapeDtypeStruct((2, NP, 128), jnp.float32),
            mesh=_sc_mesh(),
            scratch_types=[
                pltpu.VMEM((2, 1, 128), jnp.int32),
                pltpu.VMEM((4, 1, 128), jnp.int32),
                pltpu.VMEM(rows_shape, jnp.float32),
                pltpu.VMEM((2, WW, 128), jnp.float32),
                pltpu.VMEM((2, 4, WW + 16), jnp.float32),
                pltpu.VMEM_SHARED((NP, 128), jnp.float32),
                pltpu.SemaphoreType.DMA,
                pltpu.SemaphoreType.DMA,
                pltpu.SemaphoreType.DMA,
                pltpu.SemaphoreType.DMA,
            ],
            compiler_params=pltpu.CompilerParams(use_tc_tiling_on_sc=False),
            name=f"sc_wagg{tw}",
        )(tab, ei_r, lg, zeros)

    return run


_sc_wagg32 = _make_wagg(32)
_sc_wagg128 = _make_wagg(128)


WINL = 256
KL = WINL // 128


def _logits_body(z, ei_r, out, src_i, dst_i, zs_v, zd_v, lgo_v, sem):
    c = lax.axis_index("c")
    s = lax.axis_index("s")
    wid = s * NCOR + c
    nw = EP // 32 // WINL
    lanes = jnp.arange(16, dtype=jnp.int32)

    def window(w, carry):
        e0 = wid * (nw * WINL) + w * WINL
        row0 = e0 // 128
        pltpu.sync_copy(ei_r.at[0, pl.ds(row0, KL)], src_i)
        pltpu.sync_copy(ei_r.at[1, pl.ds(row0, KL)], dst_i)
        descs = []
        for j in range(KL):
            descs.append(pltpu.async_copy(
                z.at[src_i.at[j]], zs_v.at[pl.ds(j * 128, 128)], sem))
            descs.append(pltpu.async_copy(
                z.at[dst_i.at[j]], zd_v.at[pl.ds(j * 128, 128)], sem))
        for dsc in descs:
            dsc.wait()

        def gblock(g, carry2):
            acc = [jnp.zeros((16,), jnp.float32) for _ in range(NCOM)]
            for j in range(16):
                e = g * 16 + j
                for k in range(NCOM):
                    a = zs_v[e, pl.ds(k * 32, 16)] * zd_v[e, pl.ds(k * 32, 16)]
                    b = (zs_v[e, pl.ds(k * 32 + 16, 16)]
                         * zd_v[e, pl.ds(k * 32 + 16, 16)])
                    sk = jnp.sum(a + b)
                    acc[k] = jnp.where(lanes == j, sk, acc[k])
            for k in range(NCOM):
                lgo_v[k, pl.ds(g * 16, 16)] = acc[k]
            return carry2

        lax.fori_loop(0, WINL // 16, gblock, 0, unroll=False)
        pltpu.sync_copy(lgo_v, out.at[:, pl.ds(e0, WINL)])
        return carry

    lax.fori_loop(0, nw, window, 0, unroll=False)


@jax.jit
def _sc_logits(z, ei_r):
    return pl.kernel(
        _logits_body,
        out_type=jax.ShapeDtypeStruct((4, EP), jnp.float32),
        mesh=_sc_mesh(),
        scratch_types=[
            pltpu.VMEM((KL, 128), jnp.int32),
            pltpu.VMEM((KL, 128), jnp.int32),
            pltpu.VMEM((WINL, 128), jnp.float32),
            pltpu.VMEM((WINL, 128), jnp.float32),
            pltpu.VMEM((4, WINL), jnp.float32),
            pltpu.SemaphoreType.DMA,
        ],
        compiler_params=pltpu.CompilerParams(use_tc_tiling_on_sc=False,
                                             needs_layout_passes=False),
        name="sc_logits",
    )(z, ei_r)


def _bn(h, g, b):
    return h * (g / jnp.sqrt(1.0 + 1e-5)) + b


_BNS = float(1.0 / (1.0 + 1e-5) ** 0.5)


def _tc1_body(x_r, W_aff_r, b_aff_r, W0a_r, W_in_r, b_in_r, z_o, y_o, xin_o):
    x = x_r[...]
    z_o[...] = jnp.maximum(
        jnp.dot(x, W_aff_r[...], preferred_element_type=jnp.float32)
        + b_aff_r[...], 0.0)
    y_o[...] = jnp.dot(x, W0a_r[...], preferred_element_type=jnp.float32)
    xin_o[...] = jnp.dot(x, W_in_r[...],
                         preferred_element_type=jnp.float32) + b_in_r[...]


@jax.jit
def _tc1(x, W_aff, b_aff, W0a, W_in, b_in):
    return pl.pallas_call(
        _tc1_body,
        out_shape=[jax.ShapeDtypeStruct((N, D), jnp.float32),
                   jax.ShapeDtypeStruct((N, CD), jnp.float32),
                   jax.ShapeDtypeStruct((N, D), jnp.float32)],
    )(x, W_aff, b_aff.reshape(1, D), W0a, W_in, b_in.reshape(1, D))


def _gin0_body(p_r, y4_r, ba_r, Wb_r, bb_r, bng_r, bnb_r, Wn_r, xin_r,
               hx_o, t_o):
    agg = p_r[0, :N] + p_r[1, :N]
    u = jnp.maximum(y4_r[...] + agg + ba_r[...], 0.0)
    hpre = jnp.dot(u, Wb_r[...], preferred_element_type=jnp.float32) \
        + bb_r[...]
    h = jnp.maximum(hpre * (bng_r[...] * _BNS) + bnb_r[...], 0.0)
    hx_o[...] = h + xin_r[...]
    t_o[...] = jnp.dot(h, Wn_r[...], preferred_element_type=jnp.float32)


@jax.jit
def _gin0(p, y, b0a, W0b, b0b, bn0_g, bn0_b, W1a, xin):
    eye4 = jnp.eye(NCOM, dtype=jnp.float32)
    y4 = jnp.tile(y, (1, NCOM))
    return pl.pallas_call(
        _gin0_body,
        out_shape=[jax.ShapeDtypeStruct((N, D), jnp.float32),
                   jax.ShapeDtypeStruct((N, D), jnp.float32)],
    )(p, y4, jnp.tile(b0a, NCOM).reshape(1, D), jnp.kron(eye4, W0b),
      jnp.tile(b0b, NCOM).reshape(1, D), jnp.tile(bn0_g, NCOM).reshape(1, D),
      jnp.tile(bn0_b, NCOM).reshape(1, D), jnp.kron(eye4, W1a), xin)


def _gin1_body(p_r, t_r, ba_r, Wb_r, bb_r, bng_r, bnb_r, hx_r, Wn_r, hw_o):
    agg = p_r[0, :N] + p_r[1, :N]
    u = jnp.maximum(t_r[...] + agg + ba_r[...], 0.0)
    hpre = jnp.dot(u, Wb_r[...], preferred_element_type=jnp.float32) \
        + bb_r[...]
    h2 = jnp.maximum(hpre * (bng_r[...] * _BNS) + bnb_r[...], 0.0)
    h = hx_r[...] + h2
    hw_o[...] = jnp.dot(h, Wn_r[...], preferred_element_type=jnp.float32)


@jax.jit
def _gin1(p, t, b1a, W1b, b1b, bn1_g, bn1_b, hx, Wc0a):
    eye4 = jnp.eye(NCOM, dtype=jnp.float32)
    return pl.pallas_call(
        _gin1_body,
        out_shape=jax.ShapeDtypeStruct((N, D), jnp.float32),
    )(p, t, jnp.tile(b1a, NCOM).reshape(1, D), jnp.kron(eye4, W1b),
      jnp.tile(b1b, NCOM).reshape(1, D), jnp.tile(bn1_g, NCOM).reshape(1, D),
      jnp.tile(bn1_b, NCOM).reshape(1, D), hx, Wc0a)


def _rep_mid_body(p_r, base_r, ba_r, Wb_r, bb_r, bng_r, bnb_r, Wn_r, gw_o):
    agg = p_r[0, :N] + p_r[1, :N]
    u = jnp.maximum(base_r[...] + agg + ba_r[...], 0.0)
    hpre = jnp.dot(u, Wb_r[...], preferred_element_type=jnp.float32) \
        + bb_r[...]
    g1 = jnp.maximum(hpre * (bng_r[...] * _BNS) + bnb_r[...], 0.0)
    gw_o[...] = jnp.dot(g1, Wn_r[...], preferred_element_type=jnp.float32)


@jax.jit
def _rep_mid(p, base, ba, Wb, bb, bng, bnb, Wn):
    return pl.pallas_call(
        _rep_mid_body,
        out_shape=jax.ShapeDtypeStruct((N, D), jnp.float32),
    )(p, base, ba.reshape(1, D), Wb, bb.reshape(1, D), bng.reshape(1, D),
      bnb.reshape(1, D), Wn)


def _rep_final_body(p_r, base_r, ba_r, Wb_r, bb_r, bng_r, bnb_r, g2_o):
    agg = p_r[0, :N] + p_r[1, :N]
    u = jnp.maximum(base_r[...] + agg + ba_r[...], 0.0)
    hpre = jnp.dot(u, Wb_r[...], preferred_element_type=jnp.float32) \
        + bb_r[...]
    g2_o[...] = jnp.maximum(hpre * (bng_r[...] * _BNS) + bnb_r[...], 0.0)


@jax.jit
def _rep_final(p, base, ba, Wb, bb, bng, bnb):
    return pl.pallas_call(
        _rep_final_body,
        out_shape=jax.ShapeDtypeStruct((N, D), jnp.float32),
    )(p, base, ba.reshape(1, D), Wb, bb.reshape(1, D), bng.reshape(1, D),
      bnb.reshape(1, D))


def kernel(x, edge_index, W_aff, b_aff, W0a, b0a, W0b, b0b, bn0_g, bn0_b,
           W1a, b1a, W1b, b1b, bn1_g, bn1_b, W_in, b_in,
           Wc0a, bc0a, Wc0b, bc0b, bnc0_g, bnc0_b,
           Wc1a, bc1a, Wc1b, bc1b, bnc1_g, bnc1_b):
    src, dst = edge_index[0], edge_index[1]

    npad = EP - E
    pad_lane = (jnp.arange(npad, dtype=jnp.int32) % 112)
    ei_r = jnp.concatenate([
        jnp.stack([src, dst]),
        jnp.stack([pad_lane, N + pad_lane]),
    ], axis=1).reshape(2, EP // 128, 128)
    zerosNP = jnp.zeros((NP, 128), jnp.float32)

    z, y, xin = _tc1(x, W_aff, b_aff, W0a, W_in, b_in)

    lg = _sc_logits(z, ei_r)

    pB = _sc_wagg32(y, ei_r, lg, zerosNP)
    hx, t = _gin0(pB, y, b0a, W0b, b0b, bn0_g, bn0_b, W1a, xin)

    pC = _sc_wagg128(t, ei_r, lg, zerosNP)
    hw = _gin1(pC, t, b1a, W1b, b1b, bn1_g, bn1_b, hx, Wc0a)

    p1 = _sc_unit_agg(hw, ei_r, zerosNP)
    gw = _rep_mid(p1, hw, bc0a, Wc0b, bc0b, bnc0_g, bnc0_b, Wc1a)

    p2 = _sc_unit_agg(gw, ei_r, zerosNP)
    g2 = _rep_final(p2, gw, bc1a, Wc1b, bc1b, bnc1_g, bnc1_b)
    return g2
